# Initial kernel scaffold; baseline (speedup 1.0000x reference)
import jax
import jax.numpy as jnp
from jax.experimental import pallas as pl


def kernel(x, edge_index, edge_attr, batch, W_rel1, W_root1, b1, g1, be1, W_rel2, W_root2, b2, g2, be2, Wf, bf):
    raise NotImplementedError("write your pallas kernel here")


# speedup vs baseline: 12.2672x; 12.2672x over previous
import functools

import jax
import jax.numpy as jnp
from jax import lax
from jax.experimental import pallas as pl
from jax.experimental.pallas import tpu as pltpu
from jax.experimental.pallas import tpu_sc as plsc

N = 10000
E = 320000
R = 8
D = 128
H = 128
C = 10
G = 16

NC = 2
NS = 16
NW = NC * NS
L = 16

KE = 80
ZB1 = 5008
ZROWS = 125
EPT_CNT = E // NS
EPW = E // NW

_MESH = plsc.VectorSubcoreMesh(core_axis_name="c", subcore_axis_name="s")

_SPLAT_DNUMS = lax.GatherDimensionNumbers(
    offset_dims=(), collapsed_slice_dims=(0,), start_index_map=(0,))


def _splat(v16, lane):
    idx = jnp.full((L, 1), lane, jnp.int32)
    return lax.gather(v16, idx, _SPLAT_DNUMS, (1,),
                      mode=lax.GatherScatterMode.PROMISE_IN_BOUNDS)


def _prep_body(src_h, dst_h, et_h, gidx_h, w_h,
               a_v, b_v, c_v, cidx_v, gi_v, w_v, ones_v, cnt_v, zb_v,
               cnt_sp, sem):
    s = lax.axis_index("s")
    cc = lax.axis_index("c")

    nrpt = (N * R) // NS

    def zfill(i, carry):
        zb_v[pl.ds(i * L, L)] = jnp.zeros((L,), jnp.float32)
        return carry

    lax.fori_loop(0, ZB1 // L, zfill, 0)
    pltpu.sync_copy(zb_v.at[pl.ds(0, nrpt)], cnt_sp.at[pl.ds(s * nrpt, nrpt)])
    for j in range(KE // L):
        ones_v[pl.ds(j * L, L)] = jnp.ones((L,), jnp.float32)
    plsc.subcore_barrier()

    def cnt_body(g, carry):
        base = s * EPT_CNT + g * KE
        pltpu.sync_copy(dst_h.at[pl.ds(base, KE)], c_v)
        pltpu.sync_copy(et_h.at[pl.ds(base, KE)], b_v)
        for j in range(KE // L):
            sl = pl.ds(j * L, L)
            cidx_v[sl] = c_v[sl] * R + b_v[sl]
        pltpu.sync_copy(ones_v, cnt_sp.at[cidx_v], add=True)
        return carry

    lax.fori_loop(0, EPT_CNT // KE, cnt_body, 0)
    plsc.subcore_barrier()

    wid = s * NC + cc

    def w_body(g, carry):
        base = wid * EPW + g * KE
        pltpu.sync_copy(src_h.at[pl.ds(base, KE)], a_v)
        pltpu.sync_copy(dst_h.at[pl.ds(base, KE)], c_v)
        pltpu.sync_copy(et_h.at[pl.ds(base, KE)], b_v)
        for j in range(KE // L):
            sl = pl.ds(j * L, L)
            t16 = b_v[sl]
            gi_v[sl] = a_v[sl] * R + t16
            cidx_v[sl] = c_v[sl] * R + t16
        pltpu.async_copy(cnt_sp.at[cidx_v], cnt_v, sem).wait()
        for j in range(KE // L):
            sl = pl.ds(j * L, L)
            w_v[sl] = 1.0 / jnp.maximum(cnt_v[sl], 1.0)
        pltpu.sync_copy(gi_v, gidx_h.at[pl.ds(base, KE)])
        pltpu.sync_copy(w_v, w_h.at[pl.ds(base, KE)])
        return carry

    lax.fori_loop(0, EPW // KE, w_body, 0)


_prep = functools.partial(
    pl.kernel,
    out_type=(jax.ShapeDtypeStruct((E,), jnp.int32),
              jax.ShapeDtypeStruct((E,), jnp.float32)),
    mesh=_MESH,
    scratch_types=[
        pltpu.VMEM((KE,), jnp.int32),
        pltpu.VMEM((KE,), jnp.int32),
        pltpu.VMEM((KE,), jnp.int32),
        pltpu.VMEM((KE,), jnp.int32),
        pltpu.VMEM((KE,), jnp.int32),
        pltpu.VMEM((KE,), jnp.float32),
        pltpu.VMEM((KE,), jnp.float32),
        pltpu.VMEM((KE,), jnp.float32),
        pltpu.VMEM((ZB1,), jnp.float32),
        pltpu.VMEM_SHARED((N * R,), jnp.float32),
        pltpu.SemaphoreType.DMA,
    ],
    compiler_params=pltpu.CompilerParams(use_tc_tiling_on_sc=False),
)(_prep_body)


def _agg_body(m_h, gidx_h, dst_h, w_h, parts_h,
              gi_v, d_v, w_v, rows_v, st_v, acc_sp, sem):
    s = lax.axis_index("s")
    cc = lax.axis_index("c")
    npt = N // NS

    def zfill(i, carry):
        for jj in range(H // L):
            st_v[i, pl.ds(jj * L, L)] = jnp.zeros((L,), jnp.float32)
        return carry

    lax.fori_loop(0, ZROWS, zfill, 0)
    for q in range(npt // ZROWS):
        pltpu.sync_copy(st_v, acc_sp.at[pl.ds(s * npt + q * ZROWS, ZROWS)])
    plsc.subcore_barrier()
    wid = s * NC + cc

    def chunk_body(g, carry):
        base = wid * EPW + g * KE
        pltpu.sync_copy(gidx_h.at[pl.ds(base, KE)], gi_v)
        pltpu.sync_copy(dst_h.at[pl.ds(base, KE)], d_v)
        pltpu.sync_copy(w_h.at[pl.ds(base, KE)], w_v)
        pltpu.async_copy(m_h.at[gi_v], rows_v, sem).wait()

        for j in range(KE // L):
            w16 = w_v[pl.ds(j * L, L)]
            for l in range(L):
                ws = _splat(w16, l)
                e = j * L + l
                for hh in range(H // L):
                    sl = pl.ds(hh * L, L)
                    rows_v[e, sl] = rows_v[e, sl] * ws
        pltpu.sync_copy(rows_v, acc_sp.at[d_v], add=True)
        return carry

    lax.fori_loop(0, EPW // KE, chunk_body, 0)
    plsc.subcore_barrier()
    for q in range(npt // ZROWS):
        row0 = s * npt + q * ZROWS
        pltpu.sync_copy(acc_sp.at[pl.ds(row0, ZROWS)], st_v)
        pltpu.sync_copy(st_v, parts_h.at[cc, pl.ds(row0, ZROWS)])


_agg = functools.partial(
    pl.kernel,
    out_type=jax.ShapeDtypeStruct((NC, N, H), jnp.float32),
    mesh=_MESH,
    scratch_types=[
        pltpu.VMEM((KE,), jnp.int32),
        pltpu.VMEM((KE,), jnp.int32),
        pltpu.VMEM((KE,), jnp.float32),
        pltpu.VMEM((KE, H), jnp.float32),
        pltpu.VMEM((ZROWS, H), jnp.float32),
        pltpu.VMEM_SHARED((N, H), jnp.float32),
        pltpu.SemaphoreType.DMA,
    ],
    compiler_params=pltpu.CompilerParams(use_tc_tiling_on_sc=False),
)(_agg_body)


BN = 1000
W1 = R * H + H


def _mm_body(x_ref, w_ref, b_ref, o_ref):
    o_ref[...] = (jnp.dot(x_ref[...], w_ref[...],
                          preferred_element_type=jnp.float32) + b_ref[...])


def _matmul(x, wcat, bcat):
    k = x.shape[1]
    m = wcat.shape[1]
    return pl.pallas_call(
        _mm_body,
        grid=(N // BN,),
        in_specs=[pl.BlockSpec((BN, k), lambda i: (i, 0)),
                  pl.BlockSpec((k, m), lambda i: (0, 0)),
                  pl.BlockSpec((1, m), lambda i: (0, 0))],
        out_specs=pl.BlockSpec((BN, m), lambda i: (i, 0)),
        out_shape=jax.ShapeDtypeStruct((N, m), jnp.float32),
    )(x, wcat, bcat)


def _stats_body(p0_ref, p1_ref, rt_ref, h_ref, mu_ref, var_ref, s_ref, ss_ref):
    i = pl.program_id(0)
    h = p0_ref[...] + p1_ref[...] + rt_ref[...]
    h_ref[...] = h
    ps = jnp.sum(h, axis=0, keepdims=True)
    pss = jnp.sum(h * h, axis=0, keepdims=True)

    @pl.when(i == 0)
    def _():
        s_ref[...] = ps
        ss_ref[...] = pss

    @pl.when(i != 0)
    def _():
        s_ref[...] = s_ref[...] + ps
        ss_ref[...] = ss_ref[...] + pss

    @pl.when(i == pl.num_programs(0) - 1)
    def _():
        mu = s_ref[...] * (1.0 / N)
        var = ss_ref[...] * (1.0 / N) - mu * mu
        mu_ref[...] = mu
        var_ref[...] = var


def _stats(p0, p1, rt):
    return pl.pallas_call(
        _stats_body,
        grid=(N // BN,),
        in_specs=[pl.BlockSpec((BN, H), lambda i: (i, 0))] * 3,
        out_specs=[pl.BlockSpec((BN, H), lambda i: (i, 0)),
                   pl.BlockSpec((1, H), lambda i: (0, 0)),
                   pl.BlockSpec((1, H), lambda i: (0, 0))],
        out_shape=[jax.ShapeDtypeStruct((N, H), jnp.float32),
                   jax.ShapeDtypeStruct((1, H), jnp.float32),
                   jax.ShapeDtypeStruct((1, H), jnp.float32)],
        scratch_shapes=[pltpu.VMEM((1, H), jnp.float32),
                        pltpu.VMEM((1, H), jnp.float32)],
    )(p0, p1, rt)


def _bnmm_body(h_ref, mu_ref, var_ref, g_ref, be_ref, w_ref, b_ref, o_ref):
    scale = g_ref[...] * lax.rsqrt(var_ref[...] + 1e-5)
    shift = be_ref[...] - mu_ref[...] * scale
    hn = jnp.maximum(h_ref[...] * scale + shift, 0.0)
    o_ref[...] = (jnp.dot(hn, w_ref[...],
                          preferred_element_type=jnp.float32) + b_ref[...])


def _bn_matmul(h, mu, var, g, be, wcat, bcat):
    m = wcat.shape[1]
    return pl.pallas_call(
        _bnmm_body,
        grid=(N // BN,),
        in_specs=[pl.BlockSpec((BN, H), lambda i: (i, 0)),
                  pl.BlockSpec((1, H), lambda i: (0, 0)),
                  pl.BlockSpec((1, H), lambda i: (0, 0)),
                  pl.BlockSpec((1, H), lambda i: (0, 0)),
                  pl.BlockSpec((1, H), lambda i: (0, 0)),
                  pl.BlockSpec((H, m), lambda i: (0, 0)),
                  pl.BlockSpec((1, m), lambda i: (0, 0))],
        out_specs=pl.BlockSpec((BN, m), lambda i: (i, 0)),
        out_shape=jax.ShapeDtypeStruct((N, m), jnp.float32),
    )(h, mu, var, g, be, wcat, bcat)


def _pool_body(h_ref, mu_ref, var_ref, g_ref, be_ref, bat_ref, wf_ref, bf_ref,
               o_ref, ps_ref, cs_ref):
    i = pl.program_id(0)
    scale = g_ref[...] * lax.rsqrt(var_ref[...] + 1e-5)
    shift = be_ref[...] - mu_ref[...] * scale
    hn = jnp.maximum(h_ref[...] * scale + shift, 0.0)
    gids = lax.broadcasted_iota(jnp.int32, (BN, G), 1)
    oh = (bat_ref[...] == gids).astype(jnp.float32)
    dn = (((0,), (0,)), ((), ()))
    ps = lax.dot_general(oh, hn, dn, preferred_element_type=jnp.float32)
    cnt = lax.dot_general(oh, jnp.ones_like(hn), dn,
                          preferred_element_type=jnp.float32)

    @pl.when(i == 0)
    def _():
        ps_ref[...] = ps
        cs_ref[...] = cnt

    @pl.when(i != 0)
    def _():
        ps_ref[...] = ps_ref[...] + ps
        cs_ref[...] = cs_ref[...] + cnt

    @pl.when(i == pl.num_programs(0) - 1)
    def _():
        pooled = ps_ref[...] / jnp.maximum(cs_ref[...], 1.0)
        o_ref[...] = (jnp.dot(pooled, wf_ref[...],
                              preferred_element_type=jnp.float32) + bf_ref[...])


def _pool(h, mu, var, g, be, batf, wf_pad, bf_pad):
    return pl.pallas_call(
        _pool_body,
        grid=(N // BN,),
        in_specs=[pl.BlockSpec((BN, H), lambda i: (i, 0)),
                  pl.BlockSpec((1, H), lambda i: (0, 0)),
                  pl.BlockSpec((1, H), lambda i: (0, 0)),
                  pl.BlockSpec((1, H), lambda i: (0, 0)),
                  pl.BlockSpec((1, H), lambda i: (0, 0)),
                  pl.BlockSpec((BN, 1), lambda i: (i, 0)),
                  pl.BlockSpec((H, 128), lambda i: (0, 0)),
                  pl.BlockSpec((1, 128), lambda i: (0, 0))],
        out_specs=pl.BlockSpec((G, 128), lambda i: (0, 0)),
        out_shape=jax.ShapeDtypeStruct((G, 128), jnp.float32),
        scratch_shapes=[pltpu.VMEM((G, 128), jnp.float32),
                        pltpu.VMEM((G, 128), jnp.float32)],
    )(h, mu, var, g, be, batf, wf_pad, bf_pad)


def kernel(x, edge_index, edge_attr, batch, W_rel1, W_root1, b1, g1, be1,
           W_rel2, W_root2, b2, g2, be2, Wf, bf):
    src = edge_index[0].astype(jnp.int32)
    dst = edge_index[1].astype(jnp.int32)
    et = edge_attr.astype(jnp.int32)
    gidx, w = _prep(src, dst, et)

    wcat1 = jnp.concatenate(
        [jnp.transpose(W_rel1, (1, 0, 2)).reshape(D, R * H), W_root1], axis=1)
    bcat1 = jnp.concatenate(
        [jnp.zeros((R * H,), jnp.float32), b1]).reshape(1, W1)
    m1 = _matmul(x, wcat1, bcat1)
    mrel1 = m1[:, :R * H].reshape(N * R, H)
    root1 = m1[:, R * H:]
    parts1 = _agg(mrel1, gidx, dst, w)
    h1pre, mu1, var1 = _stats(parts1[0], parts1[1], root1)

    wcat2 = jnp.concatenate(
        [jnp.transpose(W_rel2, (1, 0, 2)).reshape(H, R * H), W_root2], axis=1)
    bcat2 = jnp.concatenate(
        [jnp.zeros((R * H,), jnp.float32), b2]).reshape(1, W1)
    m2 = _bn_matmul(h1pre, mu1, var1, g1.reshape(1, H), be1.reshape(1, H),
                    wcat2, bcat2)
    mrel2 = m2[:, :R * H].reshape(N * R, H)
    root2 = m2[:, R * H:]
    parts2 = _agg(mrel2, gidx, dst, w)
    h2pre, mu2, var2 = _stats(parts2[0], parts2[1], root2)

    batf = batch.astype(jnp.int32).reshape(N, 1)
    wf_pad = jnp.zeros((H, 128), jnp.float32).at[:, :C].set(Wf)
    bf_pad = jnp.zeros((1, 128), jnp.float32).at[0, :C].set(bf)
    outp = _pool(h2pre, mu2, var2, g2.reshape(1, H), be2.reshape(1, H),
                 batf, wf_pad, bf_pad)
    return outp[:, :C]


# speedup vs baseline: 31.5131x; 2.5689x over previous
import functools

import jax
import jax.numpy as jnp
from jax import lax
from jax.experimental import pallas as pl
from jax.experimental.pallas import tpu as pltpu
from jax.experimental.pallas import tpu_sc as plsc

N = 10000
E = 320000
R = 8
D = 128
H = 128
C = 10
G = 16

NC = 2
NS = 16
NW = NC * NS
L = 16

KE = 80
ZB1 = 5008
ZROWS = 125
EPT_CNT = E // NS
EPW = E // NW

_MESH = plsc.VectorSubcoreMesh(core_axis_name="c", subcore_axis_name="s")

_SPLAT_DNUMS = lax.GatherDimensionNumbers(
    offset_dims=(), collapsed_slice_dims=(0,), start_index_map=(0,))


def _splat(v16, lane):
    idx = jnp.full((L, 1), lane, jnp.int32)
    return lax.gather(v16, idx, _SPLAT_DNUMS, (1,),
                      mode=lax.GatherScatterMode.PROMISE_IN_BOUNDS)


NBC = EPT_CNT // KE
NBW = EPW // KE


def _prep_body(src_h, dst_h, et_h, gidx_h, w_h,
               cdst, cet, cidx, ones_v, zb_v,
               wsrc, wdst, wet, wgi, wcid, wcv, ww,
               cnt_sp, clsem, cssem, wlsem, wgsem, wssem):
    s = lax.axis_index("s")
    cc = lax.axis_index("c")

    nrpt = (N * R) // NS

    def zfill(i, carry):
        zb_v[pl.ds(i * L, L)] = jnp.zeros((L,), jnp.float32)
        return carry

    lax.fori_loop(0, ZB1 // L, zfill, 0)
    pltpu.sync_copy(zb_v.at[pl.ds(0, nrpt)], cnt_sp.at[pl.ds(s * nrpt, nrpt)])
    for j in range(KE // L):
        ones_v[pl.ds(j * L, L)] = jnp.ones((L,), jnp.float32)
    plsc.subcore_barrier()

    cbase0 = s * EPT_CNT

    def c_load(g):
        m = lax.rem(g, 2)
        base = cbase0 + g * KE
        pltpu.async_copy(dst_h.at[pl.ds(base, KE)], cdst.at[m], clsem.at[m])
        pltpu.async_copy(et_h.at[pl.ds(base, KE)], cet.at[m], clsem.at[m])

    def c_lwait(g):
        m = lax.rem(g, 2)
        pltpu.make_async_copy(dst_h.at[pl.ds(0, KE)], cdst.at[m],
                              clsem.at[m]).wait()
        pltpu.make_async_copy(et_h.at[pl.ds(0, KE)], cet.at[m],
                              clsem.at[m]).wait()

    def c_swait(g):
        m3 = lax.rem(g, 3)
        pltpu.make_async_copy(ones_v, cnt_sp.at[cidx.at[m3]],
                              cssem.at[m3]).wait()

    c_load(0)

    def cbody(g, carry):
        @pl.when(g + 1 < NBC)
        def _():
            c_load(g + 1)

        c_lwait(g)
        m2 = lax.rem(g, 2)
        m3 = lax.rem(g, 3)
        dr = cdst.at[m2]
        er = cet.at[m2]
        ci = cidx.at[m3]
        for j in range(KE // L):
            sl = pl.ds(j * L, L)
            ci[sl] = dr[sl] * R + er[sl]

        @pl.when(g >= 2)
        def _():
            c_swait(g - 2)

        pltpu.async_copy(ones_v, cnt_sp.at[cidx.at[m3]], cssem.at[m3],
                         add=True)
        return carry

    lax.fori_loop(0, NBC, cbody, 0)
    c_swait(NBC - 2)
    c_swait(NBC - 1)
    plsc.subcore_barrier()

    wid = s * NC + cc
    wbase0 = wid * EPW

    def w_load(g):
        m = lax.rem(g, 2)
        base = wbase0 + g * KE
        pltpu.async_copy(src_h.at[pl.ds(base, KE)], wsrc.at[m], wlsem.at[m])
        pltpu.async_copy(dst_h.at[pl.ds(base, KE)], wdst.at[m], wlsem.at[m])
        pltpu.async_copy(et_h.at[pl.ds(base, KE)], wet.at[m], wlsem.at[m])

    def w_lwait(g):
        m = lax.rem(g, 2)
        pltpu.make_async_copy(src_h.at[pl.ds(0, KE)], wsrc.at[m],
                              wlsem.at[m]).wait()
        pltpu.make_async_copy(dst_h.at[pl.ds(0, KE)], wdst.at[m],
                              wlsem.at[m]).wait()
        pltpu.make_async_copy(et_h.at[pl.ds(0, KE)], wet.at[m],
                              wlsem.at[m]).wait()

    def g_issue(g):
        m = lax.rem(g, 2)
        pltpu.async_copy(cnt_sp.at[wcid.at[m]], wcv.at[m], wgsem.at[m])

    def g_wait(g):
        m = lax.rem(g, 2)
        pltpu.make_async_copy(cnt_sp.at[wcid.at[m]], wcv.at[m],
                              wgsem.at[m]).wait()

    def s_issue(g):
        m3 = lax.rem(g, 3)
        base = wbase0 + g * KE
        pltpu.async_copy(wgi.at[m3], gidx_h.at[pl.ds(base, KE)],
                         wssem.at[m3])
        pltpu.async_copy(ww.at[m3], w_h.at[pl.ds(base, KE)], wssem.at[m3])

    def s_wait(g):
        m3 = lax.rem(g, 3)
        pltpu.make_async_copy(wgi.at[m3], gidx_h.at[pl.ds(0, KE)],
                              wssem.at[m3]).wait()
        pltpu.make_async_copy(ww.at[m3], w_h.at[pl.ds(0, KE)],
                              wssem.at[m3]).wait()

    def c2_and_store(g):
        m3 = lax.rem(g, 3)
        m2 = lax.rem(g, 2)
        vr = wcv.at[m2]
        wr = ww.at[m3]
        for j in range(KE // L):
            sl = pl.ds(j * L, L)
            wr[sl] = 1.0 / jnp.maximum(vr[sl], 1.0)
        s_issue(g)

    w_load(0)

    def wbody(g, carry):
        @pl.when(g + 1 < NBW)
        def _():
            w_load(g + 1)

        @pl.when(g >= 2)
        def _():
            s_wait(g - 2)

        w_lwait(g)
        m2 = lax.rem(g, 2)
        m3 = lax.rem(g, 3)
        sr = wsrc.at[m2]
        dr = wdst.at[m2]
        er = wet.at[m2]
        gr = wgi.at[m3]
        cr = wcid.at[m2]
        for j in range(KE // L):
            sl = pl.ds(j * L, L)
            t16 = er[sl]
            gr[sl] = sr[sl] * R + t16
            cr[sl] = dr[sl] * R + t16
        g_issue(g)

        @pl.when(g >= 1)
        def _():
            g_wait(g - 1)
            c2_and_store(g - 1)

        return carry

    lax.fori_loop(0, NBW, wbody, 0)
    g_wait(NBW - 1)
    c2_and_store(NBW - 1)
    s_wait(NBW - 2)
    s_wait(NBW - 1)


_prep = functools.partial(
    pl.kernel,
    out_type=(jax.ShapeDtypeStruct((E,), jnp.int32),
              jax.ShapeDtypeStruct((E,), jnp.float32)),
    mesh=_MESH,
    scratch_types=[
        pltpu.VMEM((2, KE), jnp.int32),
        pltpu.VMEM((2, KE), jnp.int32),
        pltpu.VMEM((3, KE), jnp.int32),
        pltpu.VMEM((KE,), jnp.float32),
        pltpu.VMEM((ZB1,), jnp.float32),
        pltpu.VMEM((2, KE), jnp.int32),
        pltpu.VMEM((2, KE), jnp.int32),
        pltpu.VMEM((2, KE), jnp.int32),
        pltpu.VMEM((3, KE), jnp.int32),
        pltpu.VMEM((2, KE), jnp.int32),
        pltpu.VMEM((2, KE), jnp.float32),
        pltpu.VMEM((3, KE), jnp.float32),
        pltpu.VMEM_SHARED((N * R,), jnp.float32),
        pltpu.SemaphoreType.DMA((2,)),
        pltpu.SemaphoreType.DMA((3,)),
        pltpu.SemaphoreType.DMA((2,)),
        pltpu.SemaphoreType.DMA((2,)),
        pltpu.SemaphoreType.DMA((3,)),
    ],
    compiler_params=pltpu.CompilerParams(use_tc_tiling_on_sc=False),
)(_prep_body)


NB_AGG = EPW // KE


def _agg_body(m_h, gidx_h, dst_h, w_h, parts_h,
              gi_v, d_v, w_v, rows_v, st_v, acc_sp, lsem, gsem, ssem):
    s = lax.axis_index("s")
    cc = lax.axis_index("c")
    npt = N // NS

    def zfill(i, carry):
        for jj in range(H // L):
            st_v[i, pl.ds(jj * L, L)] = jnp.zeros((L,), jnp.float32)
        return carry

    lax.fori_loop(0, ZROWS, zfill, 0)
    for q in range(npt // ZROWS):
        pltpu.sync_copy(st_v, acc_sp.at[pl.ds(s * npt + q * ZROWS, ZROWS)])
    plsc.subcore_barrier()
    wid = s * NC + cc
    ebase = wid * EPW

    def lin_issue(g):
        m = lax.rem(g, 3)
        m4 = lax.rem(g, 4)
        base = ebase + g * KE
        pltpu.async_copy(gidx_h.at[pl.ds(base, KE)], gi_v.at[m], lsem.at[m])
        pltpu.async_copy(w_h.at[pl.ds(base, KE)], w_v.at[m], lsem.at[m])
        pltpu.async_copy(dst_h.at[pl.ds(base, KE)], d_v.at[m4], lsem.at[m])

    def lin_wait(g):
        m = lax.rem(g, 3)
        m4 = lax.rem(g, 4)
        pltpu.make_async_copy(gidx_h.at[pl.ds(0, KE)], gi_v.at[m],
                              lsem.at[m]).wait()
        pltpu.make_async_copy(w_h.at[pl.ds(0, KE)], w_v.at[m],
                              lsem.at[m]).wait()
        pltpu.make_async_copy(dst_h.at[pl.ds(0, KE)], d_v.at[m4],
                              lsem.at[m]).wait()

    def gat_issue(g):
        b = lax.rem(g, 3)
        pltpu.async_copy(m_h.at[gi_v.at[b]], rows_v.at[b], gsem.at[b])

    def gat_wait(g):
        b = lax.rem(g, 3)
        pltpu.make_async_copy(m_h.at[gi_v.at[b]], rows_v.at[b],
                              gsem.at[b]).wait()

    def sct_issue(g):
        b = lax.rem(g, 3)
        m4 = lax.rem(g, 4)
        pltpu.async_copy(rows_v.at[b], acc_sp.at[d_v.at[m4]], ssem.at[b],
                         add=True)

    def sct_wait(g):
        b = lax.rem(g, 3)
        m4 = lax.rem(g, 4)
        pltpu.make_async_copy(rows_v.at[b], acc_sp.at[d_v.at[m4]],
                              ssem.at[b]).wait()

    def scale(g):
        b = lax.rem(g, 3)
        rr = rows_v.at[b]
        wr = w_v.at[b]
        for j in range(KE // L):
            w16 = wr[pl.ds(j * L, L)]
            for l in range(L):
                ws = _splat(w16, l)
                e = j * L + l
                for hh in range(H // L):
                    sl = pl.ds(hh * L, L)
                    rr[e, sl] = rr[e, sl] * ws

    lin_issue(0)
    lin_issue(1)
    lin_wait(0)
    gat_issue(0)

    def body(g, carry):
        @pl.when(g >= 2)
        def _():
            sct_wait(g - 2)

        @pl.when(g + 1 < NB_AGG)
        def _():
            lin_wait(g + 1)
            gat_issue(g + 1)

        @pl.when(g + 2 < NB_AGG)
        def _():
            lin_issue(g + 2)

        gat_wait(g)
        scale(g)
        sct_issue(g)
        return carry

    lax.fori_loop(0, NB_AGG, body, 0)
    sct_wait(NB_AGG - 2)
    sct_wait(NB_AGG - 1)
    plsc.subcore_barrier()
    for q in range(npt // ZROWS):
        row0 = s * npt + q * ZROWS
        pltpu.sync_copy(acc_sp.at[pl.ds(row0, ZROWS)], st_v)
        pltpu.sync_copy(st_v, parts_h.at[cc, pl.ds(row0, ZROWS)])


_agg = functools.partial(
    pl.kernel,
    out_type=jax.ShapeDtypeStruct((NC, N, H), jnp.float32),
    mesh=_MESH,
    scratch_types=[
        pltpu.VMEM((3, KE), jnp.int32),
        pltpu.VMEM((4, KE), jnp.int32),
        pltpu.VMEM((3, KE), jnp.float32),
        pltpu.VMEM((3, KE, H), jnp.float32),
        pltpu.VMEM((ZROWS, H), jnp.float32),
        pltpu.VMEM_SHARED((N, H), jnp.float32),
        pltpu.SemaphoreType.DMA((3,)),
        pltpu.SemaphoreType.DMA((3,)),
        pltpu.SemaphoreType.DMA((3,)),
    ],
    compiler_params=pltpu.CompilerParams(use_tc_tiling_on_sc=False),
)(_agg_body)


BN = 1000
W1 = R * H + H


def _mm_body(x_ref, w_ref, b_ref, o_ref):
    o_ref[...] = (jnp.dot(x_ref[...], w_ref[...],
                          preferred_element_type=jnp.float32) + b_ref[...])


def _matmul(x, wcat, bcat):
    k = x.shape[1]
    m = wcat.shape[1]
    return pl.pallas_call(
        _mm_body,
        grid=(N // BN,),
        in_specs=[pl.BlockSpec((BN, k), lambda i: (i, 0)),
                  pl.BlockSpec((k, m), lambda i: (0, 0)),
                  pl.BlockSpec((1, m), lambda i: (0, 0))],
        out_specs=pl.BlockSpec((BN, m), lambda i: (i, 0)),
        out_shape=jax.ShapeDtypeStruct((N, m), jnp.float32),
    )(x, wcat, bcat)


def _stats_body(p0_ref, p1_ref, rt_ref, h_ref, mu_ref, var_ref, s_ref, ss_ref):
    i = pl.program_id(0)
    h = p0_ref[...] + p1_ref[...] + rt_ref[...]
    h_ref[...] = h
    ps = jnp.sum(h, axis=0, keepdims=True)
    pss = jnp.sum(h * h, axis=0, keepdims=True)

    @pl.when(i == 0)
    def _():
        s_ref[...] = ps
        ss_ref[...] = pss

    @pl.when(i != 0)
    def _():
        s_ref[...] = s_ref[...] + ps
        ss_ref[...] = ss_ref[...] + pss

    @pl.when(i == pl.num_programs(0) - 1)
    def _():
        mu = s_ref[...] * (1.0 / N)
        var = ss_ref[...] * (1.0 / N) - mu * mu
        mu_ref[...] = mu
        var_ref[...] = var


def _stats(p0, p1, rt):
    return pl.pallas_call(
        _stats_body,
        grid=(N // BN,),
        in_specs=[pl.BlockSpec((BN, H), lambda i: (i, 0))] * 3,
        out_specs=[pl.BlockSpec((BN, H), lambda i: (i, 0)),
                   pl.BlockSpec((1, H), lambda i: (0, 0)),
                   pl.BlockSpec((1, H), lambda i: (0, 0))],
        out_shape=[jax.ShapeDtypeStruct((N, H), jnp.float32),
                   jax.ShapeDtypeStruct((1, H), jnp.float32),
                   jax.ShapeDtypeStruct((1, H), jnp.float32)],
        scratch_shapes=[pltpu.VMEM((1, H), jnp.float32),
                        pltpu.VMEM((1, H), jnp.float32)],
    )(p0, p1, rt)


def _bnmm_body(h_ref, mu_ref, var_ref, g_ref, be_ref, w_ref, b_ref, o_ref):
    scale = g_ref[...] * lax.rsqrt(var_ref[...] + 1e-5)
    shift = be_ref[...] - mu_ref[...] * scale
    hn = jnp.maximum(h_ref[...] * scale + shift, 0.0)
    o_ref[...] = (jnp.dot(hn, w_ref[...],
                          preferred_element_type=jnp.float32) + b_ref[...])


def _bn_matmul(h, mu, var, g, be, wcat, bcat):
    m = wcat.shape[1]
    return pl.pallas_call(
        _bnmm_body,
        grid=(N // BN,),
        in_specs=[pl.BlockSpec((BN, H), lambda i: (i, 0)),
                  pl.BlockSpec((1, H), lambda i: (0, 0)),
                  pl.BlockSpec((1, H), lambda i: (0, 0)),
                  pl.BlockSpec((1, H), lambda i: (0, 0)),
                  pl.BlockSpec((1, H), lambda i: (0, 0)),
                  pl.BlockSpec((H, m), lambda i: (0, 0)),
                  pl.BlockSpec((1, m), lambda i: (0, 0))],
        out_specs=pl.BlockSpec((BN, m), lambda i: (i, 0)),
        out_shape=jax.ShapeDtypeStruct((N, m), jnp.float32),
    )(h, mu, var, g, be, wcat, bcat)


def _pool_body(h_ref, mu_ref, var_ref, g_ref, be_ref, bat_ref, wf_ref, bf_ref,
               o_ref, ps_ref, cs_ref):
    i = pl.program_id(0)
    scale = g_ref[...] * lax.rsqrt(var_ref[...] + 1e-5)
    shift = be_ref[...] - mu_ref[...] * scale
    hn = jnp.maximum(h_ref[...] * scale + shift, 0.0)
    gids = lax.broadcasted_iota(jnp.int32, (BN, G), 1)
    oh = (bat_ref[...] == gids).astype(jnp.float32)
    dn = (((0,), (0,)), ((), ()))
    ps = lax.dot_general(oh, hn, dn, preferred_element_type=jnp.float32)
    cnt = lax.dot_general(oh, jnp.ones_like(hn), dn,
                          preferred_element_type=jnp.float32)

    @pl.when(i == 0)
    def _():
        ps_ref[...] = ps
        cs_ref[...] = cnt

    @pl.when(i != 0)
    def _():
        ps_ref[...] = ps_ref[...] + ps
        cs_ref[...] = cs_ref[...] + cnt

    @pl.when(i == pl.num_programs(0) - 1)
    def _():
        pooled = ps_ref[...] / jnp.maximum(cs_ref[...], 1.0)
        o_ref[...] = (jnp.dot(pooled, wf_ref[...],
                              preferred_element_type=jnp.float32) + bf_ref[...])


def _pool(h, mu, var, g, be, batf, wf_pad, bf_pad):
    return pl.pallas_call(
        _pool_body,
        grid=(N // BN,),
        in_specs=[pl.BlockSpec((BN, H), lambda i: (i, 0)),
                  pl.BlockSpec((1, H), lambda i: (0, 0)),
                  pl.BlockSpec((1, H), lambda i: (0, 0)),
                  pl.BlockSpec((1, H), lambda i: (0, 0)),
                  pl.BlockSpec((1, H), lambda i: (0, 0)),
                  pl.BlockSpec((BN, 1), lambda i: (i, 0)),
                  pl.BlockSpec((H, 128), lambda i: (0, 0)),
                  pl.BlockSpec((1, 128), lambda i: (0, 0))],
        out_specs=pl.BlockSpec((G, 128), lambda i: (0, 0)),
        out_shape=jax.ShapeDtypeStruct((G, 128), jnp.float32),
        scratch_shapes=[pltpu.VMEM((G, 128), jnp.float32),
                        pltpu.VMEM((G, 128), jnp.float32)],
    )(h, mu, var, g, be, batf, wf_pad, bf_pad)


def kernel(x, edge_index, edge_attr, batch, W_rel1, W_root1, b1, g1, be1,
           W_rel2, W_root2, b2, g2, be2, Wf, bf):
    src = edge_index[0].astype(jnp.int32)
    dst = edge_index[1].astype(jnp.int32)
    et = edge_attr.astype(jnp.int32)
    gidx, w = _prep(src, dst, et)

    wcat1 = jnp.concatenate(
        [jnp.transpose(W_rel1, (1, 0, 2)).reshape(D, R * H), W_root1], axis=1)
    bcat1 = jnp.concatenate(
        [jnp.zeros((R * H,), jnp.float32), b1]).reshape(1, W1)
    m1 = _matmul(x, wcat1, bcat1)
    mrel1 = m1[:, :R * H].reshape(N * R, H)
    root1 = m1[:, R * H:]
    parts1 = _agg(mrel1, gidx, dst, w)
    h1pre, mu1, var1 = _stats(parts1[0], parts1[1], root1)

    wcat2 = jnp.concatenate(
        [jnp.transpose(W_rel2, (1, 0, 2)).reshape(H, R * H), W_root2], axis=1)
    bcat2 = jnp.concatenate(
        [jnp.zeros((R * H,), jnp.float32), b2]).reshape(1, W1)
    m2 = _bn_matmul(h1pre, mu1, var1, g1.reshape(1, H), be1.reshape(1, H),
                    wcat2, bcat2)
    mrel2 = m2[:, :R * H].reshape(N * R, H)
    root2 = m2[:, R * H:]
    parts2 = _agg(mrel2, gidx, dst, w)
    h2pre, mu2, var2 = _stats(parts2[0], parts2[1], root2)

    batf = batch.astype(jnp.int32).reshape(N, 1)
    wf_pad = jnp.zeros((H, 128), jnp.float32).at[:, :C].set(Wf)
    bf_pad = jnp.zeros((1, 128), jnp.float32).at[0, :C].set(bf)
    outp = _pool(h2pre, mu2, var2, g2.reshape(1, H), be2.reshape(1, H),
                 batf, wf_pad, bf_pad)
    return outp[:, :C]


# speedup vs baseline: 34.9974x; 1.1106x over previous
import functools

import jax
import jax.numpy as jnp
from jax import lax
from jax.experimental import pallas as pl
from jax.experimental.pallas import tpu as pltpu
from jax.experimental.pallas import tpu_sc as plsc

N = 10000
E = 320000
R = 8
D = 128
H = 128
C = 10
G = 16

NC = 2
NS = 16
NW = NC * NS
L = 16

KE = 80
ZB1 = 5008
ZROWS = 125
EPT_CNT = E // NS
EPW = E // NW

_MESH = plsc.VectorSubcoreMesh(core_axis_name="c", subcore_axis_name="s")

_SPLAT_DNUMS = lax.GatherDimensionNumbers(
    offset_dims=(), collapsed_slice_dims=(0,), start_index_map=(0,))


def _splat(v16, lane):
    idx = jnp.full((L, 1), lane, jnp.int32)
    return lax.gather(v16, idx, _SPLAT_DNUMS, (1,),
                      mode=lax.GatherScatterMode.PROMISE_IN_BOUNDS)


NBC = EPT_CNT // KE
NBW = EPW // KE


def _prep_body(src_h, dst_h, et_h, gidx_h, w_h,
               cdst, cet, cidx, ones_v, zb_v,
               wsrc, wdst, wet, wgi, wcid, wcv, ww,
               cnt_sp, clsem, cssem, wlsem, wgsem, wssem):
    s = lax.axis_index("s")
    cc = lax.axis_index("c")

    nrpt = (N * R) // NS

    def zfill(i, carry):
        zb_v[pl.ds(i * L, L)] = jnp.zeros((L,), jnp.float32)
        return carry

    lax.fori_loop(0, ZB1 // L, zfill, 0)
    pltpu.sync_copy(zb_v.at[pl.ds(0, nrpt)], cnt_sp.at[pl.ds(s * nrpt, nrpt)])
    for j in range(KE // L):
        ones_v[pl.ds(j * L, L)] = jnp.ones((L,), jnp.float32)
    plsc.subcore_barrier()

    cbase0 = s * EPT_CNT

    def c_load(g):
        m = lax.rem(g, 2)
        base = cbase0 + g * KE
        pltpu.async_copy(dst_h.at[pl.ds(base, KE)], cdst.at[m], clsem.at[m])
        pltpu.async_copy(et_h.at[pl.ds(base, KE)], cet.at[m], clsem.at[m])

    def c_lwait(g):
        m = lax.rem(g, 2)
        pltpu.make_async_copy(dst_h.at[pl.ds(0, KE)], cdst.at[m],
                              clsem.at[m]).wait()
        pltpu.make_async_copy(et_h.at[pl.ds(0, KE)], cet.at[m],
                              clsem.at[m]).wait()

    def c_swait(g):
        m3 = lax.rem(g, 3)
        pltpu.make_async_copy(ones_v, cnt_sp.at[cidx.at[m3]],
                              cssem.at[m3]).wait()

    c_load(0)

    def cbody(g, carry):
        @pl.when(g + 1 < NBC)
        def _():
            c_load(g + 1)

        c_lwait(g)
        m2 = lax.rem(g, 2)
        m3 = lax.rem(g, 3)
        dr = cdst.at[m2]
        er = cet.at[m2]
        ci = cidx.at[m3]
        for j in range(KE // L):
            sl = pl.ds(j * L, L)
            ci[sl] = dr[sl] * R + er[sl]

        @pl.when(g >= 2)
        def _():
            c_swait(g - 2)

        pltpu.async_copy(ones_v, cnt_sp.at[cidx.at[m3]], cssem.at[m3],
                         add=True)
        return carry

    lax.fori_loop(0, NBC, cbody, 0)
    c_swait(NBC - 2)
    c_swait(NBC - 1)
    plsc.subcore_barrier()

    wid = s * NC + cc
    wbase0 = wid * EPW

    def w_load(g):
        m = lax.rem(g, 2)
        base = wbase0 + g * KE
        pltpu.async_copy(src_h.at[pl.ds(base, KE)], wsrc.at[m], wlsem.at[m])
        pltpu.async_copy(dst_h.at[pl.ds(base, KE)], wdst.at[m], wlsem.at[m])
        pltpu.async_copy(et_h.at[pl.ds(base, KE)], wet.at[m], wlsem.at[m])

    def w_lwait(g):
        m = lax.rem(g, 2)
        pltpu.make_async_copy(src_h.at[pl.ds(0, KE)], wsrc.at[m],
                              wlsem.at[m]).wait()
        pltpu.make_async_copy(dst_h.at[pl.ds(0, KE)], wdst.at[m],
                              wlsem.at[m]).wait()
        pltpu.make_async_copy(et_h.at[pl.ds(0, KE)], wet.at[m],
                              wlsem.at[m]).wait()

    def g_issue(g):
        m = lax.rem(g, 2)
        pltpu.async_copy(cnt_sp.at[wcid.at[m]], wcv.at[m], wgsem.at[m])

    def g_wait(g):
        m = lax.rem(g, 2)
        pltpu.make_async_copy(cnt_sp.at[wcid.at[m]], wcv.at[m],
                              wgsem.at[m]).wait()

    def s_issue(g):
        m3 = lax.rem(g, 3)
        base = wbase0 + g * KE
        pltpu.async_copy(wgi.at[m3], gidx_h.at[pl.ds(base, KE)],
                         wssem.at[m3])
        pltpu.async_copy(ww.at[m3], w_h.at[pl.ds(base, KE)], wssem.at[m3])

    def s_wait(g):
        m3 = lax.rem(g, 3)
        pltpu.make_async_copy(wgi.at[m3], gidx_h.at[pl.ds(0, KE)],
                              wssem.at[m3]).wait()
        pltpu.make_async_copy(ww.at[m3], w_h.at[pl.ds(0, KE)],
                              wssem.at[m3]).wait()

    def c2_and_store(g):
        m3 = lax.rem(g, 3)
        m2 = lax.rem(g, 2)
        vr = wcv.at[m2]
        wr = ww.at[m3]
        for j in range(KE // L):
            sl = pl.ds(j * L, L)
            wr[sl] = 1.0 / jnp.maximum(vr[sl], 1.0)
        s_issue(g)

    w_load(0)

    def wbody(g, carry):
        @pl.when(g + 1 < NBW)
        def _():
            w_load(g + 1)

        @pl.when(g >= 2)
        def _():
            s_wait(g - 2)

        w_lwait(g)
        m2 = lax.rem(g, 2)
        m3 = lax.rem(g, 3)
        sr = wsrc.at[m2]
        dr = wdst.at[m2]
        er = wet.at[m2]
        gr = wgi.at[m3]
        cr = wcid.at[m2]
        for j in range(KE // L):
            sl = pl.ds(j * L, L)
            t16 = er[sl]
            gr[sl] = sr[sl] * R + t16
            cr[sl] = dr[sl] * R + t16
        g_issue(g)

        @pl.when(g >= 1)
        def _():
            g_wait(g - 1)
            c2_and_store(g - 1)

        return carry

    lax.fori_loop(0, NBW, wbody, 0)
    g_wait(NBW - 1)
    c2_and_store(NBW - 1)
    s_wait(NBW - 2)
    s_wait(NBW - 1)


_prep = functools.partial(
    pl.kernel,
    out_type=(jax.ShapeDtypeStruct((E,), jnp.int32),
              jax.ShapeDtypeStruct((E,), jnp.float32)),
    mesh=_MESH,
    scratch_types=[
        pltpu.VMEM((2, KE), jnp.int32),
        pltpu.VMEM((2, KE), jnp.int32),
        pltpu.VMEM((3, KE), jnp.int32),
        pltpu.VMEM((KE,), jnp.float32),
        pltpu.VMEM((ZB1,), jnp.float32),
        pltpu.VMEM((2, KE), jnp.int32),
        pltpu.VMEM((2, KE), jnp.int32),
        pltpu.VMEM((2, KE), jnp.int32),
        pltpu.VMEM((3, KE), jnp.int32),
        pltpu.VMEM((2, KE), jnp.int32),
        pltpu.VMEM((2, KE), jnp.float32),
        pltpu.VMEM((3, KE), jnp.float32),
        pltpu.VMEM_SHARED((N * R,), jnp.float32),
        pltpu.SemaphoreType.DMA((2,)),
        pltpu.SemaphoreType.DMA((3,)),
        pltpu.SemaphoreType.DMA((2,)),
        pltpu.SemaphoreType.DMA((2,)),
        pltpu.SemaphoreType.DMA((3,)),
    ],
    compiler_params=pltpu.CompilerParams(use_tc_tiling_on_sc=False),
)(_prep_body)


NB_AGG = EPW // KE


def _agg_body(m_h, gidx_h, dst_h, w_h, parts_h,
              gi_v, d_v, w_v, rows_v, st_v, acc_sp, lsem, gsem, ssem):
    s = lax.axis_index("s")
    cc = lax.axis_index("c")
    npt = N // NS

    def zfill(i, carry):
        for jj in range(H // L):
            st_v[i, pl.ds(jj * L, L)] = jnp.zeros((L,), jnp.float32)
        return carry

    lax.fori_loop(0, ZROWS, zfill, 0)
    for q in range(npt // ZROWS):
        pltpu.sync_copy(st_v, acc_sp.at[pl.ds(s * npt + q * ZROWS, ZROWS)])
    plsc.subcore_barrier()
    wid = s * NC + cc
    ebase = wid * EPW

    def lin_issue(g):
        m = lax.rem(g, 3)
        m4 = lax.rem(g, 4)
        base = ebase + g * KE
        pltpu.async_copy(gidx_h.at[pl.ds(base, KE)], gi_v.at[m], lsem.at[m])
        pltpu.async_copy(w_h.at[pl.ds(base, KE)], w_v.at[m], lsem.at[m])
        pltpu.async_copy(dst_h.at[pl.ds(base, KE)], d_v.at[m4], lsem.at[m])

    def lin_wait(g):
        m = lax.rem(g, 3)
        m4 = lax.rem(g, 4)
        pltpu.make_async_copy(gidx_h.at[pl.ds(0, KE)], gi_v.at[m],
                              lsem.at[m]).wait()
        pltpu.make_async_copy(w_h.at[pl.ds(0, KE)], w_v.at[m],
                              lsem.at[m]).wait()
        pltpu.make_async_copy(dst_h.at[pl.ds(0, KE)], d_v.at[m4],
                              lsem.at[m]).wait()

    def gat_issue(g):
        b = lax.rem(g, 3)
        pltpu.async_copy(m_h.at[gi_v.at[b]], rows_v.at[b], gsem.at[b])

    def gat_wait(g):
        b = lax.rem(g, 3)
        pltpu.make_async_copy(m_h.at[gi_v.at[b]], rows_v.at[b],
                              gsem.at[b]).wait()

    def sct_issue(g):
        b = lax.rem(g, 3)
        m4 = lax.rem(g, 4)
        pltpu.async_copy(rows_v.at[b], acc_sp.at[d_v.at[m4]], ssem.at[b],
                         add=True)

    def sct_wait(g):
        b = lax.rem(g, 3)
        m4 = lax.rem(g, 4)
        pltpu.make_async_copy(rows_v.at[b], acc_sp.at[d_v.at[m4]],
                              ssem.at[b]).wait()

    def scale(g):
        b = lax.rem(g, 3)
        rr = rows_v.at[b]
        wr = w_v.at[b]
        for j in range(KE // L):
            w16 = wr[pl.ds(j * L, L)]
            for l in range(L):
                ws = _splat(w16, l)
                e = j * L + l
                for hh in range(H // L):
                    sl = pl.ds(hh * L, L)
                    rr[e, sl] = rr[e, sl] * ws

    lin_issue(0)
    lin_issue(1)
    lin_wait(0)
    gat_issue(0)

    def body(g, carry):
        @pl.when(g >= 2)
        def _():
            sct_wait(g - 2)

        @pl.when(g + 1 < NB_AGG)
        def _():
            lin_wait(g + 1)
            gat_issue(g + 1)

        @pl.when(g + 2 < NB_AGG)
        def _():
            lin_issue(g + 2)

        gat_wait(g)
        scale(g)
        sct_issue(g)
        return carry

    lax.fori_loop(0, NB_AGG, body, 0)
    sct_wait(NB_AGG - 2)
    sct_wait(NB_AGG - 1)
    plsc.subcore_barrier()
    for q in range(npt // ZROWS):
        row0 = s * npt + q * ZROWS
        pltpu.sync_copy(acc_sp.at[pl.ds(row0, ZROWS)], st_v)
        pltpu.sync_copy(st_v, parts_h.at[cc, pl.ds(row0, ZROWS)])


_agg = functools.partial(
    pl.kernel,
    out_type=jax.ShapeDtypeStruct((NC, N, H), jnp.float32),
    mesh=_MESH,
    scratch_types=[
        pltpu.VMEM((3, KE), jnp.int32),
        pltpu.VMEM((4, KE), jnp.int32),
        pltpu.VMEM((3, KE), jnp.float32),
        pltpu.VMEM((3, KE, H), jnp.float32),
        pltpu.VMEM((ZROWS, H), jnp.float32),
        pltpu.VMEM_SHARED((N, H), jnp.float32),
        pltpu.SemaphoreType.DMA((3,)),
        pltpu.SemaphoreType.DMA((3,)),
        pltpu.SemaphoreType.DMA((3,)),
    ],
    compiler_params=pltpu.CompilerParams(use_tc_tiling_on_sc=False),
)(_agg_body)


BN = 1000
W1 = R * H + H


def _mm_body(x_ref, w_ref, b_ref, o1_ref, o2_ref):
    res = (jnp.dot(x_ref[...], w_ref[...],
                   preferred_element_type=jnp.float32) + b_ref[...])
    o1_ref[...] = res[:, :R * H]
    o2_ref[...] = res[:, R * H:]


def _matmul(x, wcat, bcat):
    k = x.shape[1]
    m = wcat.shape[1]
    return pl.pallas_call(
        _mm_body,
        grid=(N // BN,),
        in_specs=[pl.BlockSpec((BN, k), lambda i: (i, 0)),
                  pl.BlockSpec((k, m), lambda i: (0, 0)),
                  pl.BlockSpec((1, m), lambda i: (0, 0))],
        out_specs=[pl.BlockSpec((BN, R * H), lambda i: (i, 0)),
                   pl.BlockSpec((BN, H), lambda i: (i, 0))],
        out_shape=[jax.ShapeDtypeStruct((N, R * H), jnp.float32),
                   jax.ShapeDtypeStruct((N, H), jnp.float32)],
    )(x, wcat, bcat)


def _stats_body(p0_ref, p1_ref, rt_ref, h_ref, mu_ref, var_ref, s_ref, ss_ref):
    i = pl.program_id(0)
    h = p0_ref[...] + p1_ref[...] + rt_ref[...]
    h_ref[...] = h
    ps = jnp.sum(h, axis=0, keepdims=True)
    pss = jnp.sum(h * h, axis=0, keepdims=True)

    @pl.when(i == 0)
    def _():
        s_ref[...] = ps
        ss_ref[...] = pss

    @pl.when(i != 0)
    def _():
        s_ref[...] = s_ref[...] + ps
        ss_ref[...] = ss_ref[...] + pss

    @pl.when(i == pl.num_programs(0) - 1)
    def _():
        mu = s_ref[...] * (1.0 / N)
        var = ss_ref[...] * (1.0 / N) - mu * mu
        mu_ref[...] = mu
        var_ref[...] = var


def _stats(p0, p1, rt):
    return pl.pallas_call(
        _stats_body,
        grid=(N // BN,),
        in_specs=[pl.BlockSpec((BN, H), lambda i: (i, 0))] * 3,
        out_specs=[pl.BlockSpec((BN, H), lambda i: (i, 0)),
                   pl.BlockSpec((1, H), lambda i: (0, 0)),
                   pl.BlockSpec((1, H), lambda i: (0, 0))],
        out_shape=[jax.ShapeDtypeStruct((N, H), jnp.float32),
                   jax.ShapeDtypeStruct((1, H), jnp.float32),
                   jax.ShapeDtypeStruct((1, H), jnp.float32)],
        scratch_shapes=[pltpu.VMEM((1, H), jnp.float32),
                        pltpu.VMEM((1, H), jnp.float32)],
    )(p0, p1, rt)


def _bnmm_body(h_ref, mu_ref, var_ref, g_ref, be_ref, w_ref, b_ref,
               o1_ref, o2_ref):
    scale = g_ref[...] * lax.rsqrt(var_ref[...] + 1e-5)
    shift = be_ref[...] - mu_ref[...] * scale
    hn = jnp.maximum(h_ref[...] * scale + shift, 0.0)
    res = (jnp.dot(hn, w_ref[...],
                   preferred_element_type=jnp.float32) + b_ref[...])
    o1_ref[...] = res[:, :R * H]
    o2_ref[...] = res[:, R * H:]


def _bn_matmul(h, mu, var, g, be, wcat, bcat):
    m = wcat.shape[1]
    return pl.pallas_call(
        _bnmm_body,
        grid=(N // BN,),
        in_specs=[pl.BlockSpec((BN, H), lambda i: (i, 0)),
                  pl.BlockSpec((1, H), lambda i: (0, 0)),
                  pl.BlockSpec((1, H), lambda i: (0, 0)),
                  pl.BlockSpec((1, H), lambda i: (0, 0)),
                  pl.BlockSpec((1, H), lambda i: (0, 0)),
                  pl.BlockSpec((H, m), lambda i: (0, 0)),
                  pl.BlockSpec((1, m), lambda i: (0, 0))],
        out_specs=[pl.BlockSpec((BN, R * H), lambda i: (i, 0)),
                   pl.BlockSpec((BN, H), lambda i: (i, 0))],
        out_shape=[jax.ShapeDtypeStruct((N, R * H), jnp.float32),
                   jax.ShapeDtypeStruct((N, H), jnp.float32)],
    )(h, mu, var, g, be, wcat, bcat)


def _pool_body(h_ref, mu_ref, var_ref, g_ref, be_ref, bat_ref, wf_ref, bf_ref,
               o_ref, ps_ref, cs_ref):
    i = pl.program_id(0)
    scale = g_ref[...] * lax.rsqrt(var_ref[...] + 1e-5)
    shift = be_ref[...] - mu_ref[...] * scale
    hn = jnp.maximum(h_ref[...] * scale + shift, 0.0)
    gids = lax.broadcasted_iota(jnp.int32, (BN, G), 1)
    oh = (bat_ref[...] == gids).astype(jnp.float32)
    dn = (((0,), (0,)), ((), ()))
    ps = lax.dot_general(oh, hn, dn, preferred_element_type=jnp.float32)
    cnt = lax.dot_general(oh, jnp.ones_like(hn), dn,
                          preferred_element_type=jnp.float32)

    @pl.when(i == 0)
    def _():
        ps_ref[...] = ps
        cs_ref[...] = cnt

    @pl.when(i != 0)
    def _():
        ps_ref[...] = ps_ref[...] + ps
        cs_ref[...] = cs_ref[...] + cnt

    @pl.when(i == pl.num_programs(0) - 1)
    def _():
        pooled = ps_ref[...] / jnp.maximum(cs_ref[...], 1.0)
        o_ref[...] = (jnp.dot(pooled, wf_ref[...],
                              preferred_element_type=jnp.float32) + bf_ref[...])


def _pool(h, mu, var, g, be, batf, wf_pad, bf_pad):
    return pl.pallas_call(
        _pool_body,
        grid=(N // BN,),
        in_specs=[pl.BlockSpec((BN, H), lambda i: (i, 0)),
                  pl.BlockSpec((1, H), lambda i: (0, 0)),
                  pl.BlockSpec((1, H), lambda i: (0, 0)),
                  pl.BlockSpec((1, H), lambda i: (0, 0)),
                  pl.BlockSpec((1, H), lambda i: (0, 0)),
                  pl.BlockSpec((BN, 1), lambda i: (i, 0)),
                  pl.BlockSpec((H, 128), lambda i: (0, 0)),
                  pl.BlockSpec((1, 128), lambda i: (0, 0))],
        out_specs=pl.BlockSpec((G, 128), lambda i: (0, 0)),
        out_shape=jax.ShapeDtypeStruct((G, 128), jnp.float32),
        scratch_shapes=[pltpu.VMEM((G, 128), jnp.float32),
                        pltpu.VMEM((G, 128), jnp.float32)],
    )(h, mu, var, g, be, batf, wf_pad, bf_pad)


def kernel(x, edge_index, edge_attr, batch, W_rel1, W_root1, b1, g1, be1,
           W_rel2, W_root2, b2, g2, be2, Wf, bf):
    src = edge_index[0].astype(jnp.int32)
    dst = edge_index[1].astype(jnp.int32)
    et = edge_attr.astype(jnp.int32)
    gidx, w = _prep(src, dst, et)

    wcat1 = jnp.concatenate(
        [jnp.transpose(W_rel1, (1, 0, 2)).reshape(D, R * H), W_root1], axis=1)
    bcat1 = jnp.concatenate(
        [jnp.zeros((R * H,), jnp.float32), b1]).reshape(1, W1)
    m1rel, root1 = _matmul(x, wcat1, bcat1)
    mrel1 = m1rel.reshape(N * R, H)
    parts1 = _agg(mrel1, gidx, dst, w)
    h1pre, mu1, var1 = _stats(parts1[0], parts1[1], root1)

    wcat2 = jnp.concatenate(
        [jnp.transpose(W_rel2, (1, 0, 2)).reshape(H, R * H), W_root2], axis=1)
    bcat2 = jnp.concatenate(
        [jnp.zeros((R * H,), jnp.float32), b2]).reshape(1, W1)
    m2rel, root2 = _bn_matmul(h1pre, mu1, var1, g1.reshape(1, H),
                              be1.reshape(1, H), wcat2, bcat2)
    mrel2 = m2rel.reshape(N * R, H)
    parts2 = _agg(mrel2, gidx, dst, w)
    h2pre, mu2, var2 = _stats(parts2[0], parts2[1], root2)

    batf = batch.astype(jnp.int32).reshape(N, 1)
    wf_pad = jnp.zeros((H, 128), jnp.float32).at[:, :C].set(Wf)
    bf_pad = jnp.zeros((1, 128), jnp.float32).at[0, :C].set(bf)
    outp = _pool(h2pre, mu2, var2, g2.reshape(1, H), be2.reshape(1, H),
                 batf, wf_pad, bf_pad)
    return outp[:, :C]


# speedup vs baseline: 37.1181x; 1.0606x over previous
import functools

import jax
import jax.numpy as jnp
from jax import lax
from jax.experimental import pallas as pl
from jax.experimental.pallas import tpu as pltpu
from jax.experimental.pallas import tpu_sc as plsc

N = 10000
E = 320000
R = 8
D = 128
H = 128
C = 10
G = 16

NC = 2
NS = 16
NW = NC * NS
L = 16

KE = 80
ZB1 = 5008
NPT = 625
ZROWS = 125
EPT_CNT = E // NS
EPW = E // NW

_MESH = plsc.VectorSubcoreMesh(core_axis_name="c", subcore_axis_name="s")

_SPLAT_DNUMS = lax.GatherDimensionNumbers(
    offset_dims=(), collapsed_slice_dims=(0,), start_index_map=(0,))


def _splat(v16, lane):
    idx = jnp.full((L, 1), lane, jnp.int32)
    return lax.gather(v16, idx, _SPLAT_DNUMS, (1,),
                      mode=lax.GatherScatterMode.PROMISE_IN_BOUNDS)


NBC = EPT_CNT // KE
NBW = EPW // KE


def _prep_body(src_h, dst_h, et_h, gidx_h, w_h,
               cdst, cet, cidx, ones_v, zb_v,
               wsrc, wdst, wet, wgi, wcid, wcv, ww,
               cnt_sp, clsem, cssem, wlsem, wgsem, wssem):
    s = lax.axis_index("s")
    cc = lax.axis_index("c")

    nrpt = (N * R) // NS

    def zfill(i, carry):
        zb_v[pl.ds(i * L, L)] = jnp.zeros((L,), jnp.float32)
        return carry

    lax.fori_loop(0, ZB1 // L, zfill, 0)
    pltpu.sync_copy(zb_v.at[pl.ds(0, nrpt)], cnt_sp.at[pl.ds(s * nrpt, nrpt)])
    for j in range(KE // L):
        ones_v[pl.ds(j * L, L)] = jnp.ones((L,), jnp.float32)
    plsc.subcore_barrier()

    cbase0 = s * EPT_CNT

    def c_load(g):
        m = lax.rem(g, 2)
        base = cbase0 + g * KE
        pltpu.async_copy(dst_h.at[pl.ds(base, KE)], cdst.at[m], clsem.at[m])
        pltpu.async_copy(et_h.at[pl.ds(base, KE)], cet.at[m], clsem.at[m])

    def c_lwait(g):
        m = lax.rem(g, 2)
        pltpu.make_async_copy(dst_h.at[pl.ds(0, KE)], cdst.at[m],
                              clsem.at[m]).wait()
        pltpu.make_async_copy(et_h.at[pl.ds(0, KE)], cet.at[m],
                              clsem.at[m]).wait()

    def c_swait(g):
        m3 = lax.rem(g, 3)
        pltpu.make_async_copy(ones_v, cnt_sp.at[cidx.at[m3]],
                              cssem.at[m3]).wait()

    c_load(0)

    def cbody(g, carry):
        @pl.when(g + 1 < NBC)
        def _():
            c_load(g + 1)

        c_lwait(g)
        m2 = lax.rem(g, 2)
        m3 = lax.rem(g, 3)
        dr = cdst.at[m2]
        er = cet.at[m2]
        ci = cidx.at[m3]
        for j in range(KE // L):
            sl = pl.ds(j * L, L)
            ci[sl] = dr[sl] * R + er[sl]

        @pl.when(g >= 2)
        def _():
            c_swait(g - 2)

        pltpu.async_copy(ones_v, cnt_sp.at[cidx.at[m3]], cssem.at[m3],
                         add=True)
        return carry

    lax.fori_loop(0, NBC, cbody, 0)
    c_swait(NBC - 2)
    c_swait(NBC - 1)
    plsc.subcore_barrier()

    wid = s * NC + cc
    wbase0 = wid * EPW

    def w_load(g):
        m = lax.rem(g, 2)
        base = wbase0 + g * KE
        pltpu.async_copy(src_h.at[pl.ds(base, KE)], wsrc.at[m], wlsem.at[m])
        pltpu.async_copy(dst_h.at[pl.ds(base, KE)], wdst.at[m], wlsem.at[m])
        pltpu.async_copy(et_h.at[pl.ds(base, KE)], wet.at[m], wlsem.at[m])

    def w_lwait(g):
        m = lax.rem(g, 2)
        pltpu.make_async_copy(src_h.at[pl.ds(0, KE)], wsrc.at[m],
                              wlsem.at[m]).wait()
        pltpu.make_async_copy(dst_h.at[pl.ds(0, KE)], wdst.at[m],
                              wlsem.at[m]).wait()
        pltpu.make_async_copy(et_h.at[pl.ds(0, KE)], wet.at[m],
                              wlsem.at[m]).wait()

    def g_issue(g):
        m = lax.rem(g, 2)
        pltpu.async_copy(cnt_sp.at[wcid.at[m]], wcv.at[m], wgsem.at[m])

    def g_wait(g):
        m = lax.rem(g, 2)
        pltpu.make_async_copy(cnt_sp.at[wcid.at[m]], wcv.at[m],
                              wgsem.at[m]).wait()

    def s_issue(g):
        m3 = lax.rem(g, 3)
        base = wbase0 + g * KE
        pltpu.async_copy(wgi.at[m3], gidx_h.at[pl.ds(base, KE)],
                         wssem.at[m3])
        pltpu.async_copy(ww.at[m3], w_h.at[pl.ds(base, KE)], wssem.at[m3])

    def s_wait(g):
        m3 = lax.rem(g, 3)
        pltpu.make_async_copy(wgi.at[m3], gidx_h.at[pl.ds(0, KE)],
                              wssem.at[m3]).wait()
        pltpu.make_async_copy(ww.at[m3], w_h.at[pl.ds(0, KE)],
                              wssem.at[m3]).wait()

    def c2_and_store(g):
        m3 = lax.rem(g, 3)
        m2 = lax.rem(g, 2)
        vr = wcv.at[m2]
        wr = ww.at[m3]
        for j in range(KE // L):
            sl = pl.ds(j * L, L)
            wr[sl] = 1.0 / jnp.maximum(vr[sl], 1.0)
        s_issue(g)

    w_load(0)

    def wbody(g, carry):
        @pl.when(g + 1 < NBW)
        def _():
            w_load(g + 1)

        @pl.when(g >= 2)
        def _():
            s_wait(g - 2)

        w_lwait(g)
        m2 = lax.rem(g, 2)
        m3 = lax.rem(g, 3)
        sr = wsrc.at[m2]
        dr = wdst.at[m2]
        er = wet.at[m2]
        gr = wgi.at[m3]
        cr = wcid.at[m2]
        for j in range(KE // L):
            sl = pl.ds(j * L, L)
            t16 = er[sl]
            gr[sl] = t16 * N + sr[sl]
            cr[sl] = dr[sl] * R + t16
        g_issue(g)

        @pl.when(g >= 1)
        def _():
            g_wait(g - 1)
            c2_and_store(g - 1)

        return carry

    lax.fori_loop(0, NBW, wbody, 0)
    g_wait(NBW - 1)
    c2_and_store(NBW - 1)
    s_wait(NBW - 2)
    s_wait(NBW - 1)


_prep = functools.partial(
    pl.kernel,
    out_type=(jax.ShapeDtypeStruct((E,), jnp.int32),
              jax.ShapeDtypeStruct((E,), jnp.float32)),
    mesh=_MESH,
    scratch_types=[
        pltpu.VMEM((2, KE), jnp.int32),
        pltpu.VMEM((2, KE), jnp.int32),
        pltpu.VMEM((3, KE), jnp.int32),
        pltpu.VMEM((KE,), jnp.float32),
        pltpu.VMEM((ZB1,), jnp.float32),
        pltpu.VMEM((2, KE), jnp.int32),
        pltpu.VMEM((2, KE), jnp.int32),
        pltpu.VMEM((2, KE), jnp.int32),
        pltpu.VMEM((3, KE), jnp.int32),
        pltpu.VMEM((2, KE), jnp.int32),
        pltpu.VMEM((2, KE), jnp.float32),
        pltpu.VMEM((3, KE), jnp.float32),
        pltpu.VMEM_SHARED((N * R,), jnp.float32),
        pltpu.SemaphoreType.DMA((2,)),
        pltpu.SemaphoreType.DMA((3,)),
        pltpu.SemaphoreType.DMA((2,)),
        pltpu.SemaphoreType.DMA((2,)),
        pltpu.SemaphoreType.DMA((3,)),
    ],
    compiler_params=pltpu.CompilerParams(use_tc_tiling_on_sc=False),
)(_prep_body)


NB_AGG = EPW // KE


def _agg_body(m_h, gidx_h, dst_h, w_h, parts_h,
              gi_v, d_v, w_v, rows_v, st_v, acc_sp, lsem, gsem, ssem):
    s = lax.axis_index("s")
    cc = lax.axis_index("c")

    def zfill(i, carry):
        for jj in range(H // L):
            st_v[i, pl.ds(jj * L, L)] = jnp.zeros((L,), jnp.float32)
        return carry

    lax.fori_loop(0, ZROWS, zfill, 0)
    for q in range(NPT // ZROWS):
        pltpu.sync_copy(st_v, acc_sp.at[pl.ds(s * NPT + q * ZROWS, ZROWS)])
    plsc.subcore_barrier()
    wid = s * NC + cc
    ebase = wid * EPW

    def lin_issue(g):
        m = lax.rem(g, 3)
        m4 = lax.rem(g, 4)
        base = ebase + g * KE
        pltpu.async_copy(gidx_h.at[pl.ds(base, KE)], gi_v.at[m], lsem.at[m])
        pltpu.async_copy(w_h.at[pl.ds(base, KE)], w_v.at[m], lsem.at[m])
        pltpu.async_copy(dst_h.at[pl.ds(base, KE)], d_v.at[m4], lsem.at[m])

    def lin_wait(g):
        m = lax.rem(g, 3)
        m4 = lax.rem(g, 4)
        pltpu.make_async_copy(gidx_h.at[pl.ds(0, KE)], gi_v.at[m],
                              lsem.at[m]).wait()
        pltpu.make_async_copy(w_h.at[pl.ds(0, KE)], w_v.at[m],
                              lsem.at[m]).wait()
        pltpu.make_async_copy(dst_h.at[pl.ds(0, KE)], d_v.at[m4],
                              lsem.at[m]).wait()

    def gat_issue(g):
        b = lax.rem(g, 3)
        pltpu.async_copy(m_h.at[gi_v.at[b]], rows_v.at[b], gsem.at[b])

    def gat_wait(g):
        b = lax.rem(g, 3)
        pltpu.make_async_copy(m_h.at[gi_v.at[b]], rows_v.at[b],
                              gsem.at[b]).wait()

    def sct_issue(g):
        b = lax.rem(g, 3)
        m4 = lax.rem(g, 4)
        pltpu.async_copy(rows_v.at[b], acc_sp.at[d_v.at[m4]], ssem.at[b],
                         add=True)

    def sct_wait(g):
        b = lax.rem(g, 3)
        m4 = lax.rem(g, 4)
        pltpu.make_async_copy(rows_v.at[b], acc_sp.at[d_v.at[m4]],
                              ssem.at[b]).wait()

    def scale(g):
        b = lax.rem(g, 3)
        rr = rows_v.at[b]
        wr = w_v.at[b]
        for j in range(KE // L):
            w16 = wr[pl.ds(j * L, L)]
            for l in range(L):
                ws = _splat(w16, l)
                e = j * L + l
                for hh in range(H // L):
                    sl = pl.ds(hh * L, L)
                    rr[e, sl] = rr[e, sl] * ws

    lin_issue(0)
    lin_issue(1)
    lin_wait(0)
    gat_issue(0)

    def body(g, carry):
        @pl.when(g >= 2)
        def _():
            sct_wait(g - 2)

        @pl.when(g + 1 < NB_AGG)
        def _():
            lin_wait(g + 1)
            gat_issue(g + 1)

        @pl.when(g + 2 < NB_AGG)
        def _():
            lin_issue(g + 2)

        gat_wait(g)
        scale(g)
        sct_issue(g)
        return carry

    lax.fori_loop(0, NB_AGG, body, 0)
    sct_wait(NB_AGG - 2)
    sct_wait(NB_AGG - 1)
    plsc.subcore_barrier()
    for q in range(NPT // ZROWS):
        row0 = s * NPT + q * ZROWS
        pltpu.sync_copy(acc_sp.at[pl.ds(row0, ZROWS)], st_v)
        pltpu.sync_copy(st_v, parts_h.at[cc, pl.ds(row0, ZROWS)])


_agg = functools.partial(
    pl.kernel,
    out_type=jax.ShapeDtypeStruct((NC, N, H), jnp.float32),
    mesh=_MESH,
    scratch_types=[
        pltpu.VMEM((3, KE), jnp.int32),
        pltpu.VMEM((4, KE), jnp.int32),
        pltpu.VMEM((3, KE), jnp.float32),
        pltpu.VMEM((3, KE, H), jnp.float32),
        pltpu.VMEM((ZROWS, H), jnp.float32),
        pltpu.VMEM_SHARED((N, H), jnp.float32),
        pltpu.SemaphoreType.DMA((3,)),
        pltpu.SemaphoreType.DMA((3,)),
        pltpu.SemaphoreType.DMA((3,)),
    ],
    compiler_params=pltpu.CompilerParams(use_tc_tiling_on_sc=False),
)(_agg_body)


BN = 1000
W1 = R * H + H


def _mmrel_body(x_ref, w_ref, o_ref):
    o_ref[0] = jnp.dot(x_ref[...], w_ref[0],
                       preferred_element_type=jnp.float32)


def _matmul_rel(x, wrel):
    return pl.pallas_call(
        _mmrel_body,
        grid=(N // BN, R),
        in_specs=[pl.BlockSpec((BN, D), lambda i, r: (i, 0)),
                  pl.BlockSpec((1, D, H), lambda i, r: (r, 0, 0))],
        out_specs=pl.BlockSpec((1, BN, H), lambda i, r: (r, i, 0)),
        out_shape=jax.ShapeDtypeStruct((R, N, H), jnp.float32),
    )(x, wrel)


def _bnmmrel_body(h_ref, mu_ref, var_ref, g_ref, be_ref, w_ref, o_ref):
    scale = g_ref[...] * lax.rsqrt(var_ref[...] + 1e-5)
    shift = be_ref[...] - mu_ref[...] * scale
    hn = jnp.maximum(h_ref[...] * scale + shift, 0.0)
    o_ref[0] = jnp.dot(hn, w_ref[0], preferred_element_type=jnp.float32)


def _bn_matmul_rel(h, mu, var, g, be, wrel):
    return pl.pallas_call(
        _bnmmrel_body,
        grid=(N // BN, R),
        in_specs=[pl.BlockSpec((BN, H), lambda i, r: (i, 0)),
                  pl.BlockSpec((1, H), lambda i, r: (0, 0)),
                  pl.BlockSpec((1, H), lambda i, r: (0, 0)),
                  pl.BlockSpec((1, H), lambda i, r: (0, 0)),
                  pl.BlockSpec((1, H), lambda i, r: (0, 0)),
                  pl.BlockSpec((1, H, H), lambda i, r: (r, 0, 0))],
        out_specs=pl.BlockSpec((1, BN, H), lambda i, r: (r, i, 0)),
        out_shape=jax.ShapeDtypeStruct((R, N, H), jnp.float32),
    )(h, mu, var, g, be, wrel)


def _stats_accum(i, h, h_ref, mu_ref, var_ref, s_ref, ss_ref):
    h_ref[...] = h
    ps = jnp.sum(h, axis=0, keepdims=True)
    pss = jnp.sum(h * h, axis=0, keepdims=True)

    @pl.when(i == 0)
    def _():
        s_ref[...] = ps
        ss_ref[...] = pss

    @pl.when(i != 0)
    def _():
        s_ref[...] = s_ref[...] + ps
        ss_ref[...] = ss_ref[...] + pss

    @pl.when(i == pl.num_programs(0) - 1)
    def _():
        mu = s_ref[...] * (1.0 / N)
        var = ss_ref[...] * (1.0 / N) - mu * mu
        mu_ref[...] = mu
        var_ref[...] = var


def _stats_root_body(p0_ref, p1_ref, x_ref, wr_ref, b_ref,
                     h_ref, mu_ref, var_ref, s_ref, ss_ref):
    i = pl.program_id(0)
    h = (p0_ref[...] + p1_ref[...] + b_ref[...]
         + jnp.dot(x_ref[...], wr_ref[...],
                   preferred_element_type=jnp.float32))
    _stats_accum(i, h, h_ref, mu_ref, var_ref, s_ref, ss_ref)


def _stats_root(p0, p1, x, wroot, b):
    return pl.pallas_call(
        _stats_root_body,
        grid=(N // BN,),
        in_specs=[pl.BlockSpec((BN, H), lambda i: (i, 0)),
                  pl.BlockSpec((BN, H), lambda i: (i, 0)),
                  pl.BlockSpec((BN, H), lambda i: (i, 0)),
                  pl.BlockSpec((H, H), lambda i: (0, 0)),
                  pl.BlockSpec((1, H), lambda i: (0, 0))],
        out_specs=[pl.BlockSpec((BN, H), lambda i: (i, 0)),
                   pl.BlockSpec((1, H), lambda i: (0, 0)),
                   pl.BlockSpec((1, H), lambda i: (0, 0))],
        out_shape=[jax.ShapeDtypeStruct((N, H), jnp.float32),
                   jax.ShapeDtypeStruct((1, H), jnp.float32),
                   jax.ShapeDtypeStruct((1, H), jnp.float32)],
        scratch_shapes=[pltpu.VMEM((1, H), jnp.float32),
                        pltpu.VMEM((1, H), jnp.float32)],
    )(p0, p1, x, wroot, b)


def _stats_bn_root_body(p0_ref, p1_ref, hp_ref, mu0_ref, var0_ref, g_ref,
                        be_ref, wr_ref, b_ref,
                        h_ref, mu_ref, var_ref, s_ref, ss_ref):
    i = pl.program_id(0)
    scale = g_ref[...] * lax.rsqrt(var0_ref[...] + 1e-5)
    shift = be_ref[...] - mu0_ref[...] * scale
    hn = jnp.maximum(hp_ref[...] * scale + shift, 0.0)
    h = (p0_ref[...] + p1_ref[...] + b_ref[...]
         + jnp.dot(hn, wr_ref[...], preferred_element_type=jnp.float32))
    _stats_accum(i, h, h_ref, mu_ref, var_ref, s_ref, ss_ref)


def _stats_bn_root(p0, p1, hpre, mu0, var0, g, be, wroot, b):
    return pl.pallas_call(
        _stats_bn_root_body,
        grid=(N // BN,),
        in_specs=[pl.BlockSpec((BN, H), lambda i: (i, 0)),
                  pl.BlockSpec((BN, H), lambda i: (i, 0)),
                  pl.BlockSpec((BN, H), lambda i: (i, 0)),
                  pl.BlockSpec((1, H), lambda i: (0, 0)),
                  pl.BlockSpec((1, H), lambda i: (0, 0)),
                  pl.BlockSpec((1, H), lambda i: (0, 0)),
                  pl.BlockSpec((1, H), lambda i: (0, 0)),
                  pl.BlockSpec((H, H), lambda i: (0, 0)),
                  pl.BlockSpec((1, H), lambda i: (0, 0))],
        out_specs=[pl.BlockSpec((BN, H), lambda i: (i, 0)),
                   pl.BlockSpec((1, H), lambda i: (0, 0)),
                   pl.BlockSpec((1, H), lambda i: (0, 0))],
        out_shape=[jax.ShapeDtypeStruct((N, H), jnp.float32),
                   jax.ShapeDtypeStruct((1, H), jnp.float32),
                   jax.ShapeDtypeStruct((1, H), jnp.float32)],
        scratch_shapes=[pltpu.VMEM((1, H), jnp.float32),
                        pltpu.VMEM((1, H), jnp.float32)],
    )(p0, p1, hpre, mu0, var0, g, be, wroot, b)


def _pool_body(h_ref, mu_ref, var_ref, g_ref, be_ref, bat_ref, wf_ref, bf_ref,
               o_ref, ps_ref, cs_ref):
    i = pl.program_id(0)
    scale = g_ref[...] * lax.rsqrt(var_ref[...] + 1e-5)
    shift = be_ref[...] - mu_ref[...] * scale
    hn = jnp.maximum(h_ref[...] * scale + shift, 0.0)
    gids = lax.broadcasted_iota(jnp.int32, (BN, G), 1)
    oh = (bat_ref[...] == gids).astype(jnp.float32)
    dn = (((0,), (0,)), ((), ()))
    ps = lax.dot_general(oh, hn, dn, preferred_element_type=jnp.float32)
    cnt = lax.dot_general(oh, jnp.ones_like(hn), dn,
                          preferred_element_type=jnp.float32)

    @pl.when(i == 0)
    def _():
        ps_ref[...] = ps
        cs_ref[...] = cnt

    @pl.when(i != 0)
    def _():
        ps_ref[...] = ps_ref[...] + ps
        cs_ref[...] = cs_ref[...] + cnt

    @pl.when(i == pl.num_programs(0) - 1)
    def _():
        pooled = ps_ref[...] / jnp.maximum(cs_ref[...], 1.0)
        o_ref[...] = (jnp.dot(pooled, wf_ref[...],
                              preferred_element_type=jnp.float32) + bf_ref[...])


def _pool(h, mu, var, g, be, batf, wf_pad, bf_pad):
    return pl.pallas_call(
        _pool_body,
        grid=(N // BN,),
        in_specs=[pl.BlockSpec((BN, H), lambda i: (i, 0)),
                  pl.BlockSpec((1, H), lambda i: (0, 0)),
                  pl.BlockSpec((1, H), lambda i: (0, 0)),
                  pl.BlockSpec((1, H), lambda i: (0, 0)),
                  pl.BlockSpec((1, H), lambda i: (0, 0)),
                  pl.BlockSpec((BN, 1), lambda i: (i, 0)),
                  pl.BlockSpec((H, 128), lambda i: (0, 0)),
                  pl.BlockSpec((1, 128), lambda i: (0, 0))],
        out_specs=pl.BlockSpec((G, 128), lambda i: (0, 0)),
        out_shape=jax.ShapeDtypeStruct((G, 128), jnp.float32),
        scratch_shapes=[pltpu.VMEM((G, 128), jnp.float32),
                        pltpu.VMEM((G, 128), jnp.float32)],
    )(h, mu, var, g, be, batf, wf_pad, bf_pad)


def kernel(x, edge_index, edge_attr, batch, W_rel1, W_root1, b1, g1, be1,
           W_rel2, W_root2, b2, g2, be2, Wf, bf):
    src = edge_index[0].astype(jnp.int32)
    dst = edge_index[1].astype(jnp.int32)
    et = edge_attr.astype(jnp.int32)
    gidx, w = _prep(src, dst, et)

    mrel1 = _matmul_rel(x, W_rel1).reshape(R * N, H)
    parts1 = _agg(mrel1, gidx, dst, w)
    h1pre, mu1, var1 = _stats_root(parts1[0], parts1[1], x, W_root1,
                                   b1.reshape(1, H))

    mrel2 = _bn_matmul_rel(h1pre, mu1, var1, g1.reshape(1, H),
                           be1.reshape(1, H), W_rel2).reshape(R * N, H)
    parts2 = _agg(mrel2, gidx, dst, w)
    h2pre, mu2, var2 = _stats_bn_root(parts2[0], parts2[1], h1pre, mu1, var1,
                                      g1.reshape(1, H), be1.reshape(1, H),
                                      W_root2, b2.reshape(1, H))

    batf = batch.astype(jnp.int32).reshape(N, 1)
    wf_pad = jnp.zeros((H, 128), jnp.float32).at[:, :C].set(Wf)
    bf_pad = jnp.zeros((1, 128), jnp.float32).at[0, :C].set(bf)
    outp = _pool(h2pre, mu2, var2, g2.reshape(1, H), be2.reshape(1, H),
                 batf, wf_pad, bf_pad)
    return outp[:, :C]


# speedup vs baseline: 37.3153x; 1.0053x over previous
import functools

import jax
import jax.numpy as jnp
from jax import lax
from jax.experimental import pallas as pl
from jax.experimental.pallas import tpu as pltpu
from jax.experimental.pallas import tpu_sc as plsc

N = 10000
E = 320000
R = 8
D = 128
H = 128
C = 10
G = 16

NC = 2
NS = 16
NW = NC * NS
L = 16

KE = 80
ZB1 = 5008
NPT = 625
ZROWS = 25
EPT_CNT = E // NS
EPW = E // NW

_MESH = plsc.VectorSubcoreMesh(core_axis_name="c", subcore_axis_name="s")

_SPLAT_DNUMS = lax.GatherDimensionNumbers(
    offset_dims=(), collapsed_slice_dims=(0,), start_index_map=(0,))


def _splat(v16, lane):
    idx = jnp.full((L, 1), lane, jnp.int32)
    return lax.gather(v16, idx, _SPLAT_DNUMS, (1,),
                      mode=lax.GatherScatterMode.PROMISE_IN_BOUNDS)


NBC = EPT_CNT // KE
NBW = EPW // KE


def _prep_body(src_h, dst_h, et_h, gidx_h, w_h,
               cdst, cet, cidx, ones_v, zb_v,
               wsrc, wdst, wet, wgi, wcid, wcv, ww,
               cnt_sp, clsem, cssem, wlsem, wgsem, wssem):
    s = lax.axis_index("s")
    cc = lax.axis_index("c")

    nrpt = (N * R) // NS

    def zfill(i, carry):
        zb_v[pl.ds(i * L, L)] = jnp.zeros((L,), jnp.float32)
        return carry

    lax.fori_loop(0, ZB1 // L, zfill, 0)
    pltpu.sync_copy(zb_v.at[pl.ds(0, nrpt)], cnt_sp.at[pl.ds(s * nrpt, nrpt)])
    for j in range(KE // L):
        ones_v[pl.ds(j * L, L)] = jnp.ones((L,), jnp.float32)
    plsc.subcore_barrier()

    cbase0 = s * EPT_CNT

    def c_load(g):
        m = lax.rem(g, 2)
        base = cbase0 + g * KE
        pltpu.async_copy(dst_h.at[pl.ds(base, KE)], cdst.at[m], clsem.at[m])
        pltpu.async_copy(et_h.at[pl.ds(base, KE)], cet.at[m], clsem.at[m])

    def c_lwait(g):
        m = lax.rem(g, 2)
        pltpu.make_async_copy(dst_h.at[pl.ds(0, KE)], cdst.at[m],
                              clsem.at[m]).wait()
        pltpu.make_async_copy(et_h.at[pl.ds(0, KE)], cet.at[m],
                              clsem.at[m]).wait()

    def c_swait(g):
        m3 = lax.rem(g, 3)
        pltpu.make_async_copy(ones_v, cnt_sp.at[cidx.at[m3]],
                              cssem.at[m3]).wait()

    c_load(0)

    def cbody(g, carry):
        @pl.when(g + 1 < NBC)
        def _():
            c_load(g + 1)

        c_lwait(g)
        m2 = lax.rem(g, 2)
        m3 = lax.rem(g, 3)
        dr = cdst.at[m2]
        er = cet.at[m2]
        ci = cidx.at[m3]
        for j in range(KE // L):
            sl = pl.ds(j * L, L)
            ci[sl] = dr[sl] * R + er[sl]

        @pl.when(g >= 2)
        def _():
            c_swait(g - 2)

        pltpu.async_copy(ones_v, cnt_sp.at[cidx.at[m3]], cssem.at[m3],
                         add=True)
        return carry

    lax.fori_loop(0, NBC, cbody, 0)
    c_swait(NBC - 2)
    c_swait(NBC - 1)
    plsc.subcore_barrier()

    wid = s * NC + cc
    wbase0 = wid * EPW

    def w_load(g):
        m = lax.rem(g, 2)
        base = wbase0 + g * KE
        pltpu.async_copy(src_h.at[pl.ds(base, KE)], wsrc.at[m], wlsem.at[m])
        pltpu.async_copy(dst_h.at[pl.ds(base, KE)], wdst.at[m], wlsem.at[m])
        pltpu.async_copy(et_h.at[pl.ds(base, KE)], wet.at[m], wlsem.at[m])

    def w_lwait(g):
        m = lax.rem(g, 2)
        pltpu.make_async_copy(src_h.at[pl.ds(0, KE)], wsrc.at[m],
                              wlsem.at[m]).wait()
        pltpu.make_async_copy(dst_h.at[pl.ds(0, KE)], wdst.at[m],
                              wlsem.at[m]).wait()
        pltpu.make_async_copy(et_h.at[pl.ds(0, KE)], wet.at[m],
                              wlsem.at[m]).wait()

    def g_issue(g):
        m = lax.rem(g, 2)
        pltpu.async_copy(cnt_sp.at[wcid.at[m]], wcv.at[m], wgsem.at[m])

    def g_wait(g):
        m = lax.rem(g, 2)
        pltpu.make_async_copy(cnt_sp.at[wcid.at[m]], wcv.at[m],
                              wgsem.at[m]).wait()

    crow0 = wid * NBW

    def s_issue(g):
        m3 = lax.rem(g, 3)
        pltpu.async_copy(wgi.at[m3], gidx_h.at[crow0 + g], wssem.at[m3])
        pltpu.async_copy(ww.at[m3], w_h.at[crow0 + g], wssem.at[m3])

    def s_wait(g):
        m3 = lax.rem(g, 3)
        pltpu.make_async_copy(wgi.at[m3], gidx_h.at[crow0],
                              wssem.at[m3]).wait()
        pltpu.make_async_copy(ww.at[m3], w_h.at[crow0],
                              wssem.at[m3]).wait()

    def c2_and_store(g):
        m3 = lax.rem(g, 3)
        m2 = lax.rem(g, 2)
        vr = wcv.at[m2]
        wr = ww.at[m3]
        for j in range(KE // L):
            sl = pl.ds(j * L, L)
            wr[sl] = 1.0 / jnp.maximum(vr[sl], 1.0)
        s_issue(g)

    w_load(0)

    def wbody(g, carry):
        @pl.when(g + 1 < NBW)
        def _():
            w_load(g + 1)

        @pl.when(g >= 2)
        def _():
            s_wait(g - 2)

        w_lwait(g)
        m2 = lax.rem(g, 2)
        m3 = lax.rem(g, 3)
        sr = wsrc.at[m2]
        dr = wdst.at[m2]
        er = wet.at[m2]
        gr = wgi.at[m3]
        cr = wcid.at[m2]
        for j in range(KE // L):
            sl = pl.ds(j * L, L)
            t16 = er[sl]
            gr[sl] = t16 * N + sr[sl]
            cr[sl] = dr[sl] * R + t16
        g_issue(g)

        @pl.when(g >= 1)
        def _():
            g_wait(g - 1)
            c2_and_store(g - 1)

        return carry

    lax.fori_loop(0, NBW, wbody, 0)
    g_wait(NBW - 1)
    c2_and_store(NBW - 1)
    s_wait(NBW - 2)
    s_wait(NBW - 1)


_prep = functools.partial(
    pl.kernel,
    out_type=(jax.ShapeDtypeStruct((E // KE, KE), jnp.int32),
              jax.ShapeDtypeStruct((E // KE, KE), jnp.float32)),
    mesh=_MESH,
    scratch_types=[
        pltpu.VMEM((2, KE), jnp.int32),
        pltpu.VMEM((2, KE), jnp.int32),
        pltpu.VMEM((3, KE), jnp.int32),
        pltpu.VMEM((KE,), jnp.float32),
        pltpu.VMEM((ZB1,), jnp.float32),
        pltpu.VMEM((2, KE), jnp.int32),
        pltpu.VMEM((2, KE), jnp.int32),
        pltpu.VMEM((2, KE), jnp.int32),
        pltpu.VMEM((3, KE), jnp.int32),
        pltpu.VMEM((2, KE), jnp.int32),
        pltpu.VMEM((2, KE), jnp.float32),
        pltpu.VMEM((3, KE), jnp.float32),
        pltpu.VMEM_SHARED((N * R,), jnp.float32),
        pltpu.SemaphoreType.DMA((2,)),
        pltpu.SemaphoreType.DMA((3,)),
        pltpu.SemaphoreType.DMA((2,)),
        pltpu.SemaphoreType.DMA((2,)),
        pltpu.SemaphoreType.DMA((3,)),
    ],
    compiler_params=pltpu.CompilerParams(use_tc_tiling_on_sc=False),
)(_prep_body)


NB_AGG = EPW // KE
SUB = 5
NSC = NB_AGG // SUB
CPW = EPW // KE


def _agg_body(m_h, gidx_h, dst_h, w_h, parts_h,
              gi_v, d_v, w_v, rows_v, st_v, acc_sp, lsem, gsem, ssem):
    s = lax.axis_index("s")
    cc = lax.axis_index("c")

    def zfill(i, carry):
        for jj in range(H // L):
            st_v[i, pl.ds(jj * L, L)] = jnp.zeros((L,), jnp.float32)
        return carry

    lax.fori_loop(0, ZROWS, zfill, 0)
    for q in range(NPT // ZROWS):
        pltpu.sync_copy(st_v, acc_sp.at[pl.ds(s * NPT + q * ZROWS, ZROWS)])
    plsc.subcore_barrier()
    wid = s * NC + cc
    cbase = wid * CPW

    def lin_issue(t):
        m = lax.rem(t, 3)
        m4 = lax.rem(t, 4)
        row = cbase + t * SUB
        pltpu.async_copy(gidx_h.at[pl.ds(row, SUB)], gi_v.at[m], lsem.at[m])
        pltpu.async_copy(w_h.at[pl.ds(row, SUB)], w_v.at[m], lsem.at[m])
        pltpu.async_copy(dst_h.at[pl.ds(row, SUB)], d_v.at[m4], lsem.at[m])

    def lin_wait(t):
        m = lax.rem(t, 3)
        m4 = lax.rem(t, 4)
        pltpu.make_async_copy(gidx_h.at[pl.ds(0, SUB)], gi_v.at[m],
                              lsem.at[m]).wait()
        pltpu.make_async_copy(w_h.at[pl.ds(0, SUB)], w_v.at[m],
                              lsem.at[m]).wait()
        pltpu.make_async_copy(dst_h.at[pl.ds(0, SUB)], d_v.at[m4],
                              lsem.at[m]).wait()

    def gat_issue(g):
        b = lax.rem(g, 4)
        m = lax.rem(lax.div(g, SUB), 3)
        k = lax.rem(g, SUB)
        pltpu.async_copy(m_h.at[gi_v.at[m, k]], rows_v.at[b], gsem.at[b])

    def gat_wait(g):
        b = lax.rem(g, 4)
        m = lax.rem(lax.div(g, SUB), 3)
        k = lax.rem(g, SUB)
        pltpu.make_async_copy(m_h.at[gi_v.at[m, k]], rows_v.at[b],
                              gsem.at[b]).wait()

    def sct_issue(g):
        b = lax.rem(g, 4)
        m4 = lax.rem(lax.div(g, SUB), 4)
        k = lax.rem(g, SUB)
        pltpu.async_copy(rows_v.at[b], acc_sp.at[d_v.at[m4, k]], ssem.at[b],
                         add=True)

    def sct_wait(g):
        b = lax.rem(g, 4)
        m4 = lax.rem(lax.div(g, SUB), 4)
        k = lax.rem(g, SUB)
        pltpu.make_async_copy(rows_v.at[b], acc_sp.at[d_v.at[m4, k]],
                              ssem.at[b]).wait()

    def scale(g):
        b = lax.rem(g, 4)
        m = lax.rem(lax.div(g, SUB), 3)
        k = lax.rem(g, SUB)
        rr = rows_v.at[b]
        wr = w_v.at[m, k]
        for j in range(KE // L):
            w16 = wr[pl.ds(j * L, L)]
            for l in range(L):
                ws = _splat(w16, l)
                e = j * L + l
                for hh in range(H // L):
                    sl = pl.ds(hh * L, L)
                    rr[e, sl] = rr[e, sl] * ws

    lin_issue(0)
    lin_issue(1)
    lin_wait(0)
    gat_issue(0)
    gat_issue(1)

    def body(g, carry):
        @pl.when(g >= 3)
        def _():
            sct_wait(g - 3)

        @pl.when(g + 2 < NB_AGG)
        def _():
            @pl.when(lax.rem(g + 2, SUB) == 0)
            def _():
                lin_wait(lax.div(g + 2, SUB))

            gat_issue(g + 2)

        gat_wait(g)

        @pl.when(jnp.logical_and(lax.rem(g, SUB) == SUB - 1,
                                 lax.div(g, SUB) + 2 < NSC))
        def _():
            lin_issue(lax.div(g, SUB) + 2)

        scale(g)
        sct_issue(g)
        return carry

    lax.fori_loop(0, NB_AGG, body, 0)
    sct_wait(NB_AGG - 3)
    sct_wait(NB_AGG - 2)
    sct_wait(NB_AGG - 1)
    plsc.subcore_barrier()
    for q in range(NPT // ZROWS):
        row0 = s * NPT + q * ZROWS
        pltpu.sync_copy(acc_sp.at[pl.ds(row0, ZROWS)], st_v)
        pltpu.sync_copy(st_v, parts_h.at[cc, pl.ds(row0, ZROWS)])


_agg = functools.partial(
    pl.kernel,
    out_type=jax.ShapeDtypeStruct((NC, N, H), jnp.float32),
    mesh=_MESH,
    scratch_types=[
        pltpu.VMEM((3, SUB, KE), jnp.int32),
        pltpu.VMEM((4, SUB, KE), jnp.int32),
        pltpu.VMEM((3, SUB, KE), jnp.float32),
        pltpu.VMEM((4, KE, H), jnp.float32),
        pltpu.VMEM((ZROWS, H), jnp.float32),
        pltpu.VMEM_SHARED((N, H), jnp.float32),
        pltpu.SemaphoreType.DMA((3,)),
        pltpu.SemaphoreType.DMA((4,)),
        pltpu.SemaphoreType.DMA((4,)),
    ],
    compiler_params=pltpu.CompilerParams(use_tc_tiling_on_sc=False),
)(_agg_body)


BN = 1000
W1 = R * H + H


def _mmrel_body(x_ref, w_ref, o_ref):
    o_ref[0] = jnp.dot(x_ref[...], w_ref[0],
                       preferred_element_type=jnp.float32)


def _matmul_rel(x, wrel):
    return pl.pallas_call(
        _mmrel_body,
        grid=(N // BN, R),
        in_specs=[pl.BlockSpec((BN, D), lambda i, r: (i, 0)),
                  pl.BlockSpec((1, D, H), lambda i, r: (r, 0, 0))],
        out_specs=pl.BlockSpec((1, BN, H), lambda i, r: (r, i, 0)),
        out_shape=jax.ShapeDtypeStruct((R, N, H), jnp.float32),
    )(x, wrel)


def _bnmmrel_body(h_ref, mu_ref, var_ref, g_ref, be_ref, w_ref, o_ref):
    scale = g_ref[...] * lax.rsqrt(var_ref[...] + 1e-5)
    shift = be_ref[...] - mu_ref[...] * scale
    hn = jnp.maximum(h_ref[...] * scale + shift, 0.0)
    o_ref[0] = jnp.dot(hn, w_ref[0], preferred_element_type=jnp.float32)


def _bn_matmul_rel(h, mu, var, g, be, wrel):
    return pl.pallas_call(
        _bnmmrel_body,
        grid=(N // BN, R),
        in_specs=[pl.BlockSpec((BN, H), lambda i, r: (i, 0)),
                  pl.BlockSpec((1, H), lambda i, r: (0, 0)),
                  pl.BlockSpec((1, H), lambda i, r: (0, 0)),
                  pl.BlockSpec((1, H), lambda i, r: (0, 0)),
                  pl.BlockSpec((1, H), lambda i, r: (0, 0)),
                  pl.BlockSpec((1, H, H), lambda i, r: (r, 0, 0))],
        out_specs=pl.BlockSpec((1, BN, H), lambda i, r: (r, i, 0)),
        out_shape=jax.ShapeDtypeStruct((R, N, H), jnp.float32),
    )(h, mu, var, g, be, wrel)


def _stats_accum(i, h, h_ref, mu_ref, var_ref, s_ref, ss_ref):
    h_ref[...] = h
    ps = jnp.sum(h, axis=0, keepdims=True)
    pss = jnp.sum(h * h, axis=0, keepdims=True)

    @pl.when(i == 0)
    def _():
        s_ref[...] = ps
        ss_ref[...] = pss

    @pl.when(i != 0)
    def _():
        s_ref[...] = s_ref[...] + ps
        ss_ref[...] = ss_ref[...] + pss

    @pl.when(i == pl.num_programs(0) - 1)
    def _():
        mu = s_ref[...] * (1.0 / N)
        var = ss_ref[...] * (1.0 / N) - mu * mu
        mu_ref[...] = mu
        var_ref[...] = var


def _stats_root_body(p0_ref, p1_ref, x_ref, wr_ref, b_ref,
                     h_ref, mu_ref, var_ref, s_ref, ss_ref):
    i = pl.program_id(0)
    h = (p0_ref[...] + p1_ref[...] + b_ref[...]
         + jnp.dot(x_ref[...], wr_ref[...],
                   preferred_element_type=jnp.float32))
    _stats_accum(i, h, h_ref, mu_ref, var_ref, s_ref, ss_ref)


def _stats_root(p0, p1, x, wroot, b):
    return pl.pallas_call(
        _stats_root_body,
        grid=(N // BN,),
        in_specs=[pl.BlockSpec((BN, H), lambda i: (i, 0)),
                  pl.BlockSpec((BN, H), lambda i: (i, 0)),
                  pl.BlockSpec((BN, H), lambda i: (i, 0)),
                  pl.BlockSpec((H, H), lambda i: (0, 0)),
                  pl.BlockSpec((1, H), lambda i: (0, 0))],
        out_specs=[pl.BlockSpec((BN, H), lambda i: (i, 0)),
                   pl.BlockSpec((1, H), lambda i: (0, 0)),
                   pl.BlockSpec((1, H), lambda i: (0, 0))],
        out_shape=[jax.ShapeDtypeStruct((N, H), jnp.float32),
                   jax.ShapeDtypeStruct((1, H), jnp.float32),
                   jax.ShapeDtypeStruct((1, H), jnp.float32)],
        scratch_shapes=[pltpu.VMEM((1, H), jnp.float32),
                        pltpu.VMEM((1, H), jnp.float32)],
    )(p0, p1, x, wroot, b)


def _stats_bn_root_body(p0_ref, p1_ref, hp_ref, mu0_ref, var0_ref, g_ref,
                        be_ref, wr_ref, b_ref,
                        h_ref, mu_ref, var_ref, s_ref, ss_ref):
    i = pl.program_id(0)
    scale = g_ref[...] * lax.rsqrt(var0_ref[...] + 1e-5)
    shift = be_ref[...] - mu0_ref[...] * scale
    hn = jnp.maximum(hp_ref[...] * scale + shift, 0.0)
    h = (p0_ref[...] + p1_ref[...] + b_ref[...]
         + jnp.dot(hn, wr_ref[...], preferred_element_type=jnp.float32))
    _stats_accum(i, h, h_ref, mu_ref, var_ref, s_ref, ss_ref)


def _stats_bn_root(p0, p1, hpre, mu0, var0, g, be, wroot, b):
    return pl.pallas_call(
        _stats_bn_root_body,
        grid=(N // BN,),
        in_specs=[pl.BlockSpec((BN, H), lambda i: (i, 0)),
                  pl.BlockSpec((BN, H), lambda i: (i, 0)),
                  pl.BlockSpec((BN, H), lambda i: (i, 0)),
                  pl.BlockSpec((1, H), lambda i: (0, 0)),
                  pl.BlockSpec((1, H), lambda i: (0, 0)),
                  pl.BlockSpec((1, H), lambda i: (0, 0)),
                  pl.BlockSpec((1, H), lambda i: (0, 0)),
                  pl.BlockSpec((H, H), lambda i: (0, 0)),
                  pl.BlockSpec((1, H), lambda i: (0, 0))],
        out_specs=[pl.BlockSpec((BN, H), lambda i: (i, 0)),
                   pl.BlockSpec((1, H), lambda i: (0, 0)),
                   pl.BlockSpec((1, H), lambda i: (0, 0))],
        out_shape=[jax.ShapeDtypeStruct((N, H), jnp.float32),
                   jax.ShapeDtypeStruct((1, H), jnp.float32),
                   jax.ShapeDtypeStruct((1, H), jnp.float32)],
        scratch_shapes=[pltpu.VMEM((1, H), jnp.float32),
                        pltpu.VMEM((1, H), jnp.float32)],
    )(p0, p1, hpre, mu0, var0, g, be, wroot, b)


def _pool_body(h_ref, mu_ref, var_ref, g_ref, be_ref, bat_ref, wf_ref, bf_ref,
               o_ref, ps_ref, cs_ref):
    i = pl.program_id(0)
    scale = g_ref[...] * lax.rsqrt(var_ref[...] + 1e-5)
    shift = be_ref[...] - mu_ref[...] * scale
    hn = jnp.maximum(h_ref[...] * scale + shift, 0.0)
    gids = lax.broadcasted_iota(jnp.int32, (BN, G), 1)
    oh = (bat_ref[...] == gids).astype(jnp.float32)
    dn = (((0,), (0,)), ((), ()))
    ps = lax.dot_general(oh, hn, dn, preferred_element_type=jnp.float32)
    cnt = lax.dot_general(oh, jnp.ones_like(hn), dn,
                          preferred_element_type=jnp.float32)

    @pl.when(i == 0)
    def _():
        ps_ref[...] = ps
        cs_ref[...] = cnt

    @pl.when(i != 0)
    def _():
        ps_ref[...] = ps_ref[...] + ps
        cs_ref[...] = cs_ref[...] + cnt

    @pl.when(i == pl.num_programs(0) - 1)
    def _():
        pooled = ps_ref[...] / jnp.maximum(cs_ref[...], 1.0)
        o_ref[...] = (jnp.dot(pooled, wf_ref[...],
                              preferred_element_type=jnp.float32) + bf_ref[...])


def _pool(h, mu, var, g, be, batf, wf_pad, bf_pad):
    return pl.pallas_call(
        _pool_body,
        grid=(N // BN,),
        in_specs=[pl.BlockSpec((BN, H), lambda i: (i, 0)),
                  pl.BlockSpec((1, H), lambda i: (0, 0)),
                  pl.BlockSpec((1, H), lambda i: (0, 0)),
                  pl.BlockSpec((1, H), lambda i: (0, 0)),
                  pl.BlockSpec((1, H), lambda i: (0, 0)),
                  pl.BlockSpec((BN, 1), lambda i: (i, 0)),
                  pl.BlockSpec((H, 128), lambda i: (0, 0)),
                  pl.BlockSpec((1, 128), lambda i: (0, 0))],
        out_specs=pl.BlockSpec((G, 128), lambda i: (0, 0)),
        out_shape=jax.ShapeDtypeStruct((G, 128), jnp.float32),
        scratch_shapes=[pltpu.VMEM((G, 128), jnp.float32),
                        pltpu.VMEM((G, 128), jnp.float32)],
    )(h, mu, var, g, be, batf, wf_pad, bf_pad)


def kernel(x, edge_index, edge_attr, batch, W_rel1, W_root1, b1, g1, be1,
           W_rel2, W_root2, b2, g2, be2, Wf, bf):
    src = edge_index[0].astype(jnp.int32)
    dst = edge_index[1].astype(jnp.int32)
    et = edge_attr.astype(jnp.int32)
    gidx, w = _prep(src, dst, et)
    dst2 = dst.reshape(E // KE, KE)

    mrel1 = _matmul_rel(x, W_rel1).reshape(R * N, H)
    parts1 = _agg(mrel1, gidx, dst2, w)
    h1pre, mu1, var1 = _stats_root(parts1[0], parts1[1], x, W_root1,
                                   b1.reshape(1, H))

    mrel2 = _bn_matmul_rel(h1pre, mu1, var1, g1.reshape(1, H),
                           be1.reshape(1, H), W_rel2).reshape(R * N, H)
    parts2 = _agg(mrel2, gidx, dst2, w)
    h2pre, mu2, var2 = _stats_bn_root(parts2[0], parts2[1], h1pre, mu1, var1,
                                      g1.reshape(1, H), be1.reshape(1, H),
                                      W_root2, b2.reshape(1, H))

    batf = batch.astype(jnp.int32).reshape(N, 1)
    wf_pad = jnp.zeros((H, 128), jnp.float32).at[:, :C].set(Wf)
    bf_pad = jnp.zeros((1, 128), jnp.float32).at[0, :C].set(bf)
    outp = _pool(h2pre, mu2, var2, g2.reshape(1, H), be2.reshape(1, H),
                 batf, wf_pad, bf_pad)
    return outp[:, :C]


# speedup vs baseline: 37.3629x; 1.0013x over previous
import functools

import jax
import jax.numpy as jnp
from jax import lax
from jax.experimental import pallas as pl
from jax.experimental.pallas import tpu as pltpu
from jax.experimental.pallas import tpu_sc as plsc

N = 10000
E = 320000
R = 8
D = 128
H = 128
C = 10
G = 16

NC = 2
NS = 16
NW = NC * NS
L = 16

KE = 80
ZB1 = 5008
NPT = 625
ZROWS = 25
EPT_CNT = E // NS
EPW = E // NW

_MESH = plsc.VectorSubcoreMesh(core_axis_name="c", subcore_axis_name="s")

_SPLAT_DNUMS = lax.GatherDimensionNumbers(
    offset_dims=(), collapsed_slice_dims=(0,), start_index_map=(0,))


def _splat(v16, lane):
    idx = jnp.full((L, 1), lane, jnp.int32)
    return lax.gather(v16, idx, _SPLAT_DNUMS, (1,),
                      mode=lax.GatherScatterMode.PROMISE_IN_BOUNDS)


NBC = EPT_CNT // KE
NBW = EPW // KE


def _prep_body(src_h, dst_h, et_h, gidx_h, w_h,
               cdst, cet, cidx, ones_v, zb_v,
               wsrc, wdst, wet, wgi, wcid, wcv, ww,
               cnt_sp, clsem, cssem, wlsem, wgsem, wssem):
    s = lax.axis_index("s")
    cc = lax.axis_index("c")

    nrpt = (N * R) // NS

    def zfill(i, carry):
        zb_v[pl.ds(i * L, L)] = jnp.zeros((L,), jnp.float32)
        return carry

    lax.fori_loop(0, ZB1 // L, zfill, 0)
    pltpu.sync_copy(zb_v.at[pl.ds(0, nrpt)], cnt_sp.at[pl.ds(s * nrpt, nrpt)])
    for j in range(KE // L):
        ones_v[pl.ds(j * L, L)] = jnp.ones((L,), jnp.float32)
    plsc.subcore_barrier()

    cbase0 = s * EPT_CNT

    def c_load(g):
        m = lax.rem(g, 2)
        base = cbase0 + g * KE
        pltpu.async_copy(dst_h.at[pl.ds(base, KE)], cdst.at[m], clsem.at[m])
        pltpu.async_copy(et_h.at[pl.ds(base, KE)], cet.at[m], clsem.at[m])

    def c_lwait(g):
        m = lax.rem(g, 2)
        pltpu.make_async_copy(dst_h.at[pl.ds(0, KE)], cdst.at[m],
                              clsem.at[m]).wait()
        pltpu.make_async_copy(et_h.at[pl.ds(0, KE)], cet.at[m],
                              clsem.at[m]).wait()

    def c_swait(g):
        m3 = lax.rem(g, 3)
        pltpu.make_async_copy(ones_v, cnt_sp.at[cidx.at[m3]],
                              cssem.at[m3]).wait()

    c_load(0)

    def cbody(g, carry):
        @pl.when(g + 1 < NBC)
        def _():
            c_load(g + 1)

        c_lwait(g)
        m2 = lax.rem(g, 2)
        m3 = lax.rem(g, 3)
        dr = cdst.at[m2]
        er = cet.at[m2]
        ci = cidx.at[m3]
        for j in range(KE // L):
            sl = pl.ds(j * L, L)
            ci[sl] = dr[sl] * R + er[sl]

        @pl.when(g >= 2)
        def _():
            c_swait(g - 2)

        pltpu.async_copy(ones_v, cnt_sp.at[cidx.at[m3]], cssem.at[m3],
                         add=True)
        return carry

    lax.fori_loop(0, NBC, cbody, 0)
    c_swait(NBC - 2)
    c_swait(NBC - 1)
    plsc.subcore_barrier()

    wid = s * NC + cc
    wbase0 = wid * EPW

    def w_load(g):
        m = lax.rem(g, 2)
        base = wbase0 + g * KE
        pltpu.async_copy(src_h.at[pl.ds(base, KE)], wsrc.at[m], wlsem.at[m])
        pltpu.async_copy(dst_h.at[pl.ds(base, KE)], wdst.at[m], wlsem.at[m])
        pltpu.async_copy(et_h.at[pl.ds(base, KE)], wet.at[m], wlsem.at[m])

    def w_lwait(g):
        m = lax.rem(g, 2)
        pltpu.make_async_copy(src_h.at[pl.ds(0, KE)], wsrc.at[m],
                              wlsem.at[m]).wait()
        pltpu.make_async_copy(dst_h.at[pl.ds(0, KE)], wdst.at[m],
                              wlsem.at[m]).wait()
        pltpu.make_async_copy(et_h.at[pl.ds(0, KE)], wet.at[m],
                              wlsem.at[m]).wait()

    def g_issue(g):
        m = lax.rem(g, 2)
        pltpu.async_copy(cnt_sp.at[wcid.at[m]], wcv.at[m], wgsem.at[m])

    def g_wait(g):
        m = lax.rem(g, 2)
        pltpu.make_async_copy(cnt_sp.at[wcid.at[m]], wcv.at[m],
                              wgsem.at[m]).wait()

    crow0 = wid * NBW

    def s_issue(g):
        m3 = lax.rem(g, 3)
        pltpu.async_copy(wgi.at[m3], gidx_h.at[crow0 + g], wssem.at[m3])
        pltpu.async_copy(ww.at[m3], w_h.at[crow0 + g], wssem.at[m3])

    def s_wait(g):
        m3 = lax.rem(g, 3)
        pltpu.make_async_copy(wgi.at[m3], gidx_h.at[crow0],
                              wssem.at[m3]).wait()
        pltpu.make_async_copy(ww.at[m3], w_h.at[crow0],
                              wssem.at[m3]).wait()

    def c2_and_store(g):
        m3 = lax.rem(g, 3)
        m2 = lax.rem(g, 2)
        vr = wcv.at[m2]
        wr = ww.at[m3]
        for j in range(KE // L):
            sl = pl.ds(j * L, L)
            wr[sl] = 1.0 / jnp.maximum(vr[sl], 1.0)
        s_issue(g)

    w_load(0)

    def wbody(g, carry):
        @pl.when(g + 1 < NBW)
        def _():
            w_load(g + 1)

        @pl.when(g >= 2)
        def _():
            s_wait(g - 2)

        w_lwait(g)
        m2 = lax.rem(g, 2)
        m3 = lax.rem(g, 3)
        sr = wsrc.at[m2]
        dr = wdst.at[m2]
        er = wet.at[m2]
        gr = wgi.at[m3]
        cr = wcid.at[m2]
        for j in range(KE // L):
            sl = pl.ds(j * L, L)
            t16 = er[sl]
            gr[sl] = t16 * N + sr[sl]
            cr[sl] = dr[sl] * R + t16
        g_issue(g)

        @pl.when(g >= 1)
        def _():
            g_wait(g - 1)
            c2_and_store(g - 1)

        return carry

    lax.fori_loop(0, NBW, wbody, 0)
    g_wait(NBW - 1)
    c2_and_store(NBW - 1)
    s_wait(NBW - 2)
    s_wait(NBW - 1)


_prep = functools.partial(
    pl.kernel,
    out_type=(jax.ShapeDtypeStruct((E // KE, KE), jnp.int32),
              jax.ShapeDtypeStruct((E // KE, KE), jnp.float32)),
    mesh=_MESH,
    scratch_types=[
        pltpu.VMEM((2, KE), jnp.int32),
        pltpu.VMEM((2, KE), jnp.int32),
        pltpu.VMEM((3, KE), jnp.int32),
        pltpu.VMEM((KE,), jnp.float32),
        pltpu.VMEM((ZB1,), jnp.float32),
        pltpu.VMEM((2, KE), jnp.int32),
        pltpu.VMEM((2, KE), jnp.int32),
        pltpu.VMEM((2, KE), jnp.int32),
        pltpu.VMEM((3, KE), jnp.int32),
        pltpu.VMEM((2, KE), jnp.int32),
        pltpu.VMEM((2, KE), jnp.float32),
        pltpu.VMEM((3, KE), jnp.float32),
        pltpu.VMEM_SHARED((N * R,), jnp.float32),
        pltpu.SemaphoreType.DMA((2,)),
        pltpu.SemaphoreType.DMA((3,)),
        pltpu.SemaphoreType.DMA((2,)),
        pltpu.SemaphoreType.DMA((2,)),
        pltpu.SemaphoreType.DMA((3,)),
    ],
    compiler_params=pltpu.CompilerParams(use_tc_tiling_on_sc=False),
)(_prep_body)


NB_AGG = EPW // KE
SUB = 5
NSC = NB_AGG // SUB
CPW = EPW // KE


def _agg_body(m_h, gidx_h, dst_h, w_h, parts_h,
              gi_v, d_v, w_v, rows_v, st_v, acc_sp, lsem, gsem, ssem):
    s = lax.axis_index("s")
    cc = lax.axis_index("c")

    def zfill(i, carry):
        for jj in range(H // L):
            st_v[i, pl.ds(jj * L, L)] = jnp.zeros((L,), jnp.float32)
        return carry

    lax.fori_loop(0, ZROWS, zfill, 0)
    for q in range(NPT // ZROWS):
        pltpu.sync_copy(st_v, acc_sp.at[pl.ds(s * NPT + q * ZROWS, ZROWS)])
    plsc.subcore_barrier()
    wid = s * NC + cc
    cbase = wid * CPW

    def lin_issue(t):
        m = lax.rem(t, 3)
        m4 = lax.rem(t, 4)
        row = cbase + t * SUB
        pltpu.async_copy(gidx_h.at[pl.ds(row, SUB)], gi_v.at[m], lsem.at[m])
        pltpu.async_copy(w_h.at[pl.ds(row, SUB)], w_v.at[m], lsem.at[m])
        pltpu.async_copy(dst_h.at[pl.ds(row, SUB)], d_v.at[m4], lsem.at[m])

    def lin_wait(t):
        m = lax.rem(t, 3)
        m4 = lax.rem(t, 4)
        pltpu.make_async_copy(gidx_h.at[pl.ds(0, SUB)], gi_v.at[m],
                              lsem.at[m]).wait()
        pltpu.make_async_copy(w_h.at[pl.ds(0, SUB)], w_v.at[m],
                              lsem.at[m]).wait()
        pltpu.make_async_copy(dst_h.at[pl.ds(0, SUB)], d_v.at[m4],
                              lsem.at[m]).wait()

    def gat_issue(g):
        b = lax.rem(g, 4)
        m = lax.rem(lax.div(g, SUB), 3)
        k = lax.rem(g, SUB)
        pltpu.async_copy(m_h.at[gi_v.at[m, k]], rows_v.at[b], gsem.at[b])

    def gat_wait(g):
        b = lax.rem(g, 4)
        m = lax.rem(lax.div(g, SUB), 3)
        k = lax.rem(g, SUB)
        pltpu.make_async_copy(m_h.at[gi_v.at[m, k]], rows_v.at[b],
                              gsem.at[b]).wait()

    def sct_issue(g):
        b = lax.rem(g, 4)
        m4 = lax.rem(lax.div(g, SUB), 4)
        k = lax.rem(g, SUB)
        pltpu.async_copy(rows_v.at[b], acc_sp.at[d_v.at[m4, k]], ssem.at[b],
                         add=True)

    def sct_wait(g):
        b = lax.rem(g, 4)
        m4 = lax.rem(lax.div(g, SUB), 4)
        k = lax.rem(g, SUB)
        pltpu.make_async_copy(rows_v.at[b], acc_sp.at[d_v.at[m4, k]],
                              ssem.at[b]).wait()

    def scale(g):
        b = lax.rem(g, 4)
        m = lax.rem(lax.div(g, SUB), 3)
        k = lax.rem(g, SUB)
        rr = rows_v.at[b]
        wr = w_v.at[m, k]
        for j in range(KE // L):
            w16 = wr[pl.ds(j * L, L)]
            for l in range(L):
                ws = _splat(w16, l)
                e = j * L + l
                for hh in range(H // L):
                    sl = pl.ds(hh * L, L)
                    rr[e, sl] = rr[e, sl] * ws

    lin_issue(0)
    lin_issue(1)
    lin_wait(0)
    gat_issue(0)
    gat_issue(1)

    def body(g, carry):
        @pl.when(g >= 2)
        def _():
            sct_wait(g - 2)

        @pl.when(g + 2 < NB_AGG)
        def _():
            @pl.when(lax.rem(g + 2, SUB) == 0)
            def _():
                lin_wait(lax.div(g + 2, SUB))

            gat_issue(g + 2)

        gat_wait(g)

        @pl.when(jnp.logical_and(lax.rem(g, SUB) == SUB - 1,
                                 lax.div(g, SUB) + 2 < NSC))
        def _():
            lin_issue(lax.div(g, SUB) + 2)

        scale(g)
        sct_issue(g)
        return carry

    lax.fori_loop(0, NB_AGG, body, 0)
    sct_wait(NB_AGG - 2)
    sct_wait(NB_AGG - 1)
    plsc.subcore_barrier()
    for q in range(NPT // ZROWS):
        row0 = s * NPT + q * ZROWS
        pltpu.sync_copy(acc_sp.at[pl.ds(row0, ZROWS)], st_v)
        pltpu.sync_copy(st_v, parts_h.at[cc, pl.ds(row0, ZROWS)])


_agg = functools.partial(
    pl.kernel,
    out_type=jax.ShapeDtypeStruct((NC, N, H), jnp.float32),
    mesh=_MESH,
    scratch_types=[
        pltpu.VMEM((3, SUB, KE), jnp.int32),
        pltpu.VMEM((4, SUB, KE), jnp.int32),
        pltpu.VMEM((3, SUB, KE), jnp.float32),
        pltpu.VMEM((4, KE, H), jnp.float32),
        pltpu.VMEM((ZROWS, H), jnp.float32),
        pltpu.VMEM_SHARED((N, H), jnp.float32),
        pltpu.SemaphoreType.DMA((3,)),
        pltpu.SemaphoreType.DMA((4,)),
        pltpu.SemaphoreType.DMA((4,)),
    ],
    compiler_params=pltpu.CompilerParams(use_tc_tiling_on_sc=False),
)(_agg_body)


BN = 1000
W1 = R * H + H


def _mmrel_body(x_ref, w_ref, o_ref):
    o_ref[0] = jnp.dot(x_ref[...], w_ref[0],
                       preferred_element_type=jnp.float32)


def _matmul_rel(x, wrel):
    return pl.pallas_call(
        _mmrel_body,
        grid=(N // BN, R),
        in_specs=[pl.BlockSpec((BN, D), lambda i, r: (i, 0)),
                  pl.BlockSpec((1, D, H), lambda i, r: (r, 0, 0))],
        out_specs=pl.BlockSpec((1, BN, H), lambda i, r: (r, i, 0)),
        out_shape=jax.ShapeDtypeStruct((R, N, H), jnp.float32),
    )(x, wrel)


def _bnmmrel_body(h_ref, mu_ref, var_ref, g_ref, be_ref, w_ref, o_ref):
    scale = g_ref[...] * lax.rsqrt(var_ref[...] + 1e-5)
    shift = be_ref[...] - mu_ref[...] * scale
    hn = jnp.maximum(h_ref[...] * scale + shift, 0.0)
    o_ref[0] = jnp.dot(hn, w_ref[0], preferred_element_type=jnp.float32)


def _bn_matmul_rel(h, mu, var, g, be, wrel):
    return pl.pallas_call(
        _bnmmrel_body,
        grid=(N // BN, R),
        in_specs=[pl.BlockSpec((BN, H), lambda i, r: (i, 0)),
                  pl.BlockSpec((1, H), lambda i, r: (0, 0)),
                  pl.BlockSpec((1, H), lambda i, r: (0, 0)),
                  pl.BlockSpec((1, H), lambda i, r: (0, 0)),
                  pl.BlockSpec((1, H), lambda i, r: (0, 0)),
                  pl.BlockSpec((1, H, H), lambda i, r: (r, 0, 0))],
        out_specs=pl.BlockSpec((1, BN, H), lambda i, r: (r, i, 0)),
        out_shape=jax.ShapeDtypeStruct((R, N, H), jnp.float32),
    )(h, mu, var, g, be, wrel)


def _stats_accum(i, h, h_ref, mu_ref, var_ref, s_ref, ss_ref):
    h_ref[...] = h
    ps = jnp.sum(h, axis=0, keepdims=True)
    pss = jnp.sum(h * h, axis=0, keepdims=True)

    @pl.when(i == 0)
    def _():
        s_ref[...] = ps
        ss_ref[...] = pss

    @pl.when(i != 0)
    def _():
        s_ref[...] = s_ref[...] + ps
        ss_ref[...] = ss_ref[...] + pss

    @pl.when(i == pl.num_programs(0) - 1)
    def _():
        mu = s_ref[...] * (1.0 / N)
        var = ss_ref[...] * (1.0 / N) - mu * mu
        mu_ref[...] = mu
        var_ref[...] = var


def _stats_root_body(p0_ref, p1_ref, x_ref, wr_ref, b_ref,
                     h_ref, mu_ref, var_ref, s_ref, ss_ref):
    i = pl.program_id(0)
    h = (p0_ref[...] + p1_ref[...] + b_ref[...]
         + jnp.dot(x_ref[...], wr_ref[...],
                   preferred_element_type=jnp.float32))
    _stats_accum(i, h, h_ref, mu_ref, var_ref, s_ref, ss_ref)


def _stats_root(p0, p1, x, wroot, b):
    return pl.pallas_call(
        _stats_root_body,
        grid=(N // BN,),
        in_specs=[pl.BlockSpec((BN, H), lambda i: (i, 0)),
                  pl.BlockSpec((BN, H), lambda i: (i, 0)),
                  pl.BlockSpec((BN, H), lambda i: (i, 0)),
                  pl.BlockSpec((H, H), lambda i: (0, 0)),
                  pl.BlockSpec((1, H), lambda i: (0, 0))],
        out_specs=[pl.BlockSpec((BN, H), lambda i: (i, 0)),
                   pl.BlockSpec((1, H), lambda i: (0, 0)),
                   pl.BlockSpec((1, H), lambda i: (0, 0))],
        out_shape=[jax.ShapeDtypeStruct((N, H), jnp.float32),
                   jax.ShapeDtypeStruct((1, H), jnp.float32),
                   jax.ShapeDtypeStruct((1, H), jnp.float32)],
        scratch_shapes=[pltpu.VMEM((1, H), jnp.float32),
                        pltpu.VMEM((1, H), jnp.float32)],
    )(p0, p1, x, wroot, b)


def _stats_bn_root_body(p0_ref, p1_ref, hp_ref, mu0_ref, var0_ref, g_ref,
                        be_ref, wr_ref, b_ref,
                        h_ref, mu_ref, var_ref, s_ref, ss_ref):
    i = pl.program_id(0)
    scale = g_ref[...] * lax.rsqrt(var0_ref[...] + 1e-5)
    shift = be_ref[...] - mu0_ref[...] * scale
    hn = jnp.maximum(hp_ref[...] * scale + shift, 0.0)
    h = (p0_ref[...] + p1_ref[...] + b_ref[...]
         + jnp.dot(hn, wr_ref[...], preferred_element_type=jnp.float32))
    _stats_accum(i, h, h_ref, mu_ref, var_ref, s_ref, ss_ref)


def _stats_bn_root(p0, p1, hpre, mu0, var0, g, be, wroot, b):
    return pl.pallas_call(
        _stats_bn_root_body,
        grid=(N // BN,),
        in_specs=[pl.BlockSpec((BN, H), lambda i: (i, 0)),
                  pl.BlockSpec((BN, H), lambda i: (i, 0)),
                  pl.BlockSpec((BN, H), lambda i: (i, 0)),
                  pl.BlockSpec((1, H), lambda i: (0, 0)),
                  pl.BlockSpec((1, H), lambda i: (0, 0)),
                  pl.BlockSpec((1, H), lambda i: (0, 0)),
                  pl.BlockSpec((1, H), lambda i: (0, 0)),
                  pl.BlockSpec((H, H), lambda i: (0, 0)),
                  pl.BlockSpec((1, H), lambda i: (0, 0))],
        out_specs=[pl.BlockSpec((BN, H), lambda i: (i, 0)),
                   pl.BlockSpec((1, H), lambda i: (0, 0)),
                   pl.BlockSpec((1, H), lambda i: (0, 0))],
        out_shape=[jax.ShapeDtypeStruct((N, H), jnp.float32),
                   jax.ShapeDtypeStruct((1, H), jnp.float32),
                   jax.ShapeDtypeStruct((1, H), jnp.float32)],
        scratch_shapes=[pltpu.VMEM((1, H), jnp.float32),
                        pltpu.VMEM((1, H), jnp.float32)],
    )(p0, p1, hpre, mu0, var0, g, be, wroot, b)


def _pool_body(h_ref, mu_ref, var_ref, g_ref, be_ref, bat_ref, wf_ref, bf_ref,
               o_ref, ps_ref, cs_ref):
    i = pl.program_id(0)
    scale = g_ref[...] * lax.rsqrt(var_ref[...] + 1e-5)
    shift = be_ref[...] - mu_ref[...] * scale
    hn = jnp.maximum(h_ref[...] * scale + shift, 0.0)
    gids = lax.broadcasted_iota(jnp.int32, (BN, G), 1)
    oh = (bat_ref[...] == gids).astype(jnp.float32)
    dn = (((0,), (0,)), ((), ()))
    ps = lax.dot_general(oh, hn, dn, preferred_element_type=jnp.float32)
    cnt = lax.dot_general(oh, jnp.ones_like(hn), dn,
                          preferred_element_type=jnp.float32)

    @pl.when(i == 0)
    def _():
        ps_ref[...] = ps
        cs_ref[...] = cnt

    @pl.when(i != 0)
    def _():
        ps_ref[...] = ps_ref[...] + ps
        cs_ref[...] = cs_ref[...] + cnt

    @pl.when(i == pl.num_programs(0) - 1)
    def _():
        pooled = ps_ref[...] / jnp.maximum(cs_ref[...], 1.0)
        o_ref[...] = (jnp.dot(pooled, wf_ref[...],
                              preferred_element_type=jnp.float32) + bf_ref[...])


def _pool(h, mu, var, g, be, batf, wf_pad, bf_pad):
    return pl.pallas_call(
        _pool_body,
        grid=(N // BN,),
        in_specs=[pl.BlockSpec((BN, H), lambda i: (i, 0)),
                  pl.BlockSpec((1, H), lambda i: (0, 0)),
                  pl.BlockSpec((1, H), lambda i: (0, 0)),
                  pl.BlockSpec((1, H), lambda i: (0, 0)),
                  pl.BlockSpec((1, H), lambda i: (0, 0)),
                  pl.BlockSpec((BN, 1), lambda i: (i, 0)),
                  pl.BlockSpec((H, 128), lambda i: (0, 0)),
                  pl.BlockSpec((1, 128), lambda i: (0, 0))],
        out_specs=pl.BlockSpec((G, 128), lambda i: (0, 0)),
        out_shape=jax.ShapeDtypeStruct((G, 128), jnp.float32),
        scratch_shapes=[pltpu.VMEM((G, 128), jnp.float32),
                        pltpu.VMEM((G, 128), jnp.float32)],
    )(h, mu, var, g, be, batf, wf_pad, bf_pad)


def kernel(x, edge_index, edge_attr, batch, W_rel1, W_root1, b1, g1, be1,
           W_rel2, W_root2, b2, g2, be2, Wf, bf):
    src = edge_index[0].astype(jnp.int32)
    dst = edge_index[1].astype(jnp.int32)
    et = edge_attr.astype(jnp.int32)
    gidx, w = _prep(src, dst, et)
    dst2 = dst.reshape(E // KE, KE)

    mrel1 = _matmul_rel(x, W_rel1).reshape(R * N, H)
    parts1 = _agg(mrel1, gidx, dst2, w)
    h1pre, mu1, var1 = _stats_root(parts1[0], parts1[1], x, W_root1,
                                   b1.reshape(1, H))

    mrel2 = _bn_matmul_rel(h1pre, mu1, var1, g1.reshape(1, H),
                           be1.reshape(1, H), W_rel2).reshape(R * N, H)
    parts2 = _agg(mrel2, gidx, dst2, w)
    h2pre, mu2, var2 = _stats_bn_root(parts2[0], parts2[1], h1pre, mu1, var1,
                                      g1.reshape(1, H), be1.reshape(1, H),
                                      W_root2, b2.reshape(1, H))

    batf = batch.astype(jnp.int32).reshape(N, 1)
    wf_pad = jnp.zeros((H, 128), jnp.float32).at[:, :C].set(Wf)
    bf_pad = jnp.zeros((1, 128), jnp.float32).at[0, :C].set(bf)
    outp = _pool(h2pre, mu2, var2, g2.reshape(1, H), be2.reshape(1, H),
                 batf, wf_pad, bf_pad)
    return outp[:, :C]
